# baseline (speedup 1.0000x reference)
import functools

import jax
import jax.numpy as jnp
from jax import lax
from jax.experimental import pallas as pl
from jax.experimental.pallas import tpu as pltpu
from jax.experimental.pallas import tpu_sc as plsc

NSC = 16
NVC = 8
F1 = NSC + 3 * NVC + 5 * NVC
C1 = float(NSC) ** -0.5
C2 = float(NSC + 2 * NVC) ** -0.5
S3 = 3.0 ** 0.5
S5 = 5.0 ** 0.5

NW = 32
CHUNK = 128


def _softplus(x):
    return jnp.maximum(x, 0.0) + jnp.log(1.0 + jnp.exp(-jnp.abs(x)))


def _sh_cols(ev):
    x = ev[:, 0:1]
    y = ev[:, 1:2]
    z = ev[:, 2:3]
    inv = 1.0 / (jnp.sqrt(x * x + y * y + z * z) + 1e-12)
    xn = x * inv
    yn = y * inv
    zn = z * inv
    sh1 = [S3 * xn, S3 * yn, S3 * zn]
    sh2 = [
        S5 * S3 * xn * zn,
        S5 * S3 * xn * yn,
        S5 * (yn * yn - 0.5 * (xn * xn + zn * zn)),
        S5 * S3 * yn * zn,
        S5 * (S3 / 2.0) * (zn * zn - xn * xn),
    ]
    return sh1, sh2


def _node_body(nf, wn, bn, ws, bs, h_ref, skip_ref):
    x = nf[...]
    h_ref[...] = jnp.dot(x, wn[...], preferred_element_type=jnp.float32) + bn[...]
    skip_ref[...] = jnp.dot(x, ws[...], preferred_element_type=jnp.float32) + bs[...]


def _edge1_body(ef, ev, xh, w1a, b1a, p1, m1_ref):
    sh1, sh2 = _sh_cols(ev[...])
    t = _softplus(
        jnp.dot(ef[...], w1a[...], preferred_element_type=jnp.float32) + b1a[...]
    )
    x = xh[...]
    a = jnp.dot(x, p1[...], preferred_element_type=jnp.float32)
    acc = a[:, 512:544]
    for k in range(16):
        acc = acc + t[:, k : k + 1] * a[:, k * 32 : (k + 1) * 32]
    pre = acc * C1
    pre0 = pre[:, 0:16]
    pre1 = pre[:, 16:24]
    pre2 = pre[:, 24:32]
    blocks = [pre0]
    for m in range(3):
        blocks.append(pre1 * sh1[m])
    for m in range(5):
        blocks.append(pre2 * sh2[m])
    m1_ref[...] = jnp.concatenate(blocks, axis=1)


def _fin1_body(o1p, cntp, h, o1_ref):
    s = o1p[0] + o1p[1]
    c = cntp[0][:, 0:1] + cntp[1][:, 0:1]
    o1 = s / jnp.maximum(c, 1.0)
    o1_ref[...] = jnp.concatenate([o1[:, 0:16] + h[...], o1[:, 16:80]], axis=1)


def _edge2_body(ef, ev, xo, w2a, b2a, p2, m2_ref):
    sh1, sh2 = _sh_cols(ev[...])
    t = _softplus(
        jnp.dot(ef[...], w2a[...], preferred_element_type=jnp.float32) + b2a[...]
    )
    f = xo[...]
    r0 = f[:, 0:16]
    r1 = (
        f[:, 16:24] * sh1[0] + f[:, 24:32] * sh1[1] + f[:, 32:40] * sh1[2]
    ) * (3.0 ** -0.5)
    r2 = (
        f[:, 40:48] * sh2[0]
        + f[:, 48:56] * sh2[1]
        + f[:, 56:64] * sh2[2]
        + f[:, 64:72] * sh2[3]
        + f[:, 72:80] * sh2[4]
    ) * (5.0 ** -0.5)
    r = jnp.concatenate([r0, r1, r2], axis=1)
    a = jnp.dot(r, p2[...], preferred_element_type=jnp.float32)
    acc = a[:, 256:272]
    for k in range(16):
        acc = acc + t[:, k : k + 1] * a[:, k * 16 : (k + 1) * 16]
    m2_ref[...] = acc * C2


def _epi_body(n_nodes, o2p, cntp, skip, gamma, beta, wnl, bnl, out_ref, stat_ref):
    ph = pl.program_id(0)
    i = pl.program_id(1)
    s = o2p[0] + o2p[1]
    c = cntp[0][:, 0:1] + cntp[1][:, 0:1]
    o2 = s / jnp.maximum(c, 1.0)

    @pl.when(jnp.logical_and(ph == 0, i == 0))
    def _():
        stat_ref[...] = jnp.zeros_like(stat_ref)

    @pl.when(ph == 0)
    def _():
        stat_ref[...] = stat_ref[...] + jnp.concatenate(
            [
                jnp.sum(o2, axis=0, keepdims=True),
                jnp.sum(o2 * o2, axis=0, keepdims=True),
            ],
            axis=0,
        )

    @pl.when(ph == 1)
    def _():
        inv_n = 1.0 / float(n_nodes)
        mu = stat_ref[0:1, :] * inv_n
        var = stat_ref[1:2, :] * inv_n - mu * mu
        xb = (o2 - mu) / jnp.sqrt(var + 1e-5) * gamma[...] + beta[...]
        y = _softplus(
            jnp.dot(_softplus(xb), wnl[...], preferred_element_type=jnp.float32)
            + bnl[...]
        )
        out_ref[...] = y + skip[...]


def _sc_gather(table, idx2d, n_rows, width):
    R = idx2d.shape[0]
    rows_per = -(-R // NW)
    mesh = plsc.VectorSubcoreMesh(core_axis_name="c", subcore_axis_name="s")

    @functools.partial(
        pl.kernel,
        out_type=jax.ShapeDtypeStruct((R * CHUNK, width), jnp.float32),
        mesh=mesh,
        scratch_types=[
            pltpu.VMEM((CHUNK,), jnp.int32),
            pltpu.VMEM((CHUNK, width), jnp.float32),
            pltpu.SemaphoreType.DMA,
        ],
        compiler_params=pltpu.CompilerParams(use_tc_tiling_on_sc=False),
    )
    def gk(table_hbm, idx_hbm, out_hbm, idx_v, rows_v, sem):
        cid = lax.axis_index("c")
        sid = lax.axis_index("s")
        wid = sid * 2 + cid

        def body(j, carry):
            row = wid * rows_per + j

            @pl.when(row < R)
            def _():
                pltpu.sync_copy(idx_hbm.at[row], idx_v)
                pltpu.async_copy(table_hbm.at[idx_v], rows_v, sem).wait()
                pltpu.sync_copy(rows_v, out_hbm.at[pl.ds(row * CHUNK, CHUNK)])

            return carry

        lax.fori_loop(0, rows_per, body, 0)

    return gk(table, idx2d)


def _sc_scatter(msg, idx2d, n_nodes, width, with_count):
    R = idx2d.shape[0]
    rows_per = -(-R // NW)
    npsub = n_nodes // 16
    mesh = plsc.VectorSubcoreMesh(core_axis_name="c", subcore_axis_name="s")

    zacc = jnp.zeros((n_nodes, width), jnp.float32)
    out_type = [jax.ShapeDtypeStruct((2, n_nodes, width), jnp.float32)]
    scratch = [
        pltpu.VMEM((CHUNK,), jnp.int32),
        pltpu.VMEM((CHUNK, width), jnp.float32),
        pltpu.VMEM_SHARED((n_nodes, width), jnp.float32),
    ]
    if with_count:
        zcnt = jnp.zeros((n_nodes, 8), jnp.float32)
        ones = jnp.ones((CHUNK, 8), jnp.float32)
        out_type.append(jax.ShapeDtypeStruct((2, n_nodes, 8), jnp.float32))
        scratch += [
            pltpu.VMEM((CHUNK, 8), jnp.float32),
            pltpu.VMEM_SHARED((n_nodes, 8), jnp.float32),
        ]
    scratch.append(pltpu.SemaphoreType.DMA)

    @functools.partial(
        pl.kernel, out_type=tuple(out_type), mesh=mesh, scratch_types=scratch,
        compiler_params=pltpu.CompilerParams(use_tc_tiling_on_sc=False),
    )
    def sk(*refs):
        if with_count:
            (m_hbm, idx_hbm, za_hbm, zc_hbm, ones_hbm, accp_hbm, cntp_hbm,
             idx_v, rows_v, acc_sh, ones_v, cnt_sh, sem) = refs
        else:
            (m_hbm, idx_hbm, za_hbm, accp_hbm,
             idx_v, rows_v, acc_sh, sem) = refs
        cid = lax.axis_index("c")
        sid = lax.axis_index("s")
        wid = sid * 2 + cid

        pltpu.sync_copy(
            za_hbm.at[pl.ds(sid * npsub, npsub)],
            acc_sh.at[pl.ds(sid * npsub, npsub)],
        )
        if with_count:
            pltpu.sync_copy(
                zc_hbm.at[pl.ds(sid * npsub, npsub)],
                cnt_sh.at[pl.ds(sid * npsub, npsub)],
            )
            pltpu.sync_copy(ones_hbm, ones_v)
        plsc.subcore_barrier()

        def body(j, carry):
            row = wid * rows_per + j

            @pl.when(row < R)
            def _():
                pltpu.sync_copy(idx_hbm.at[row], idx_v)
                pltpu.sync_copy(m_hbm.at[pl.ds(row * CHUNK, CHUNK)], rows_v)
                pltpu.sync_copy(rows_v, acc_sh.at[idx_v], add=True)
                if with_count:
                    pltpu.sync_copy(ones_v, cnt_sh.at[idx_v], add=True)

            return carry

        lax.fori_loop(0, rows_per, body, 0)
        plsc.subcore_barrier()

        pltpu.sync_copy(
            acc_sh.at[pl.ds(sid * npsub, npsub)],
            accp_hbm.at[cid, pl.ds(sid * npsub, npsub)],
        )
        if with_count:
            pltpu.sync_copy(
                cnt_sh.at[pl.ds(sid * npsub, npsub)],
                cntp_hbm.at[cid, pl.ds(sid * npsub, npsub)],
            )

    if with_count:
        return sk(msg, idx2d, zacc, zcnt, ones)
    return sk(msg, idx2d, zacc)[0]


def _build_p1(fc1_W2, fc1_b2):
    w0 = fc1_W2[:, :256].reshape(16, 16, 16)
    w1 = fc1_W2[:, 256:384].reshape(16, 16, 8)
    w2 = fc1_W2[:, 384:512].reshape(16, 16, 8)
    p = jnp.concatenate([w0, w1, w2], axis=2)
    p = p.transpose(1, 0, 2).reshape(16, 512)
    b = jnp.concatenate(
        [
            fc1_b2[:256].reshape(16, 16),
            fc1_b2[256:384].reshape(16, 8),
            fc1_b2[384:512].reshape(16, 8),
        ],
        axis=1,
    )
    return jnp.concatenate([p, b], axis=1)


def _build_p2(fc2_W2, fc2_b2):
    w00 = fc2_W2[:, :256].reshape(16, 16, 16)
    w11 = fc2_W2[:, 256:384].reshape(16, 8, 16)
    w22 = fc2_W2[:, 384:512].reshape(16, 8, 16)
    p = jnp.concatenate([w00, w11, w22], axis=1)
    p = p.transpose(1, 0, 2).reshape(32, 256)
    b = jnp.concatenate(
        [
            fc2_b2[:256].reshape(16, 16),
            fc2_b2[256:384].reshape(8, 16),
            fc2_b2[384:512].reshape(8, 16),
        ],
        axis=0,
    )
    return jnp.concatenate([p, b], axis=1)


def kernel(node_feature, edge_index, edge_feature, edge_vec, W_node, b_node,
           W_skip, b_skip, fc1_W1, fc1_b1, fc1_W2, fc1_b2, fc2_W1, fc2_b1,
           fc2_W2, fc2_b2, bn_gamma, bn_beta, W_nl2, b_nl2):
    N, D = node_feature.shape
    E = edge_index.shape[1]
    R = E // CHUNK

    src2d = edge_index[0].reshape(R, CHUNK).astype(jnp.int32)
    dst2d = edge_index[1].reshape(R, CHUNK).astype(jnp.int32)

    p1 = _build_p1(fc1_W2, fc1_b2)
    p2 = _build_p2(fc2_W2, fc2_b2)
    bn2 = b_node.reshape(1, NSC)
    bs2 = b_skip.reshape(1, D)
    b1a = fc1_b1.reshape(1, 16)
    b2a = fc2_b1.reshape(1, 16)
    gamma = bn_gamma.reshape(1, NSC)
    beta = bn_beta.reshape(1, NSC)
    bnl = b_nl2.reshape(1, D)

    BN_ = 1000
    h, skip = pl.pallas_call(
        _node_body,
        grid=(N // BN_,),
        in_specs=[
            pl.BlockSpec((BN_, D), lambda i: (i, 0)),
            pl.BlockSpec((D, NSC), lambda i: (0, 0)),
            pl.BlockSpec((1, NSC), lambda i: (0, 0)),
            pl.BlockSpec((D, D), lambda i: (0, 0)),
            pl.BlockSpec((1, D), lambda i: (0, 0)),
        ],
        out_specs=[
            pl.BlockSpec((BN_, NSC), lambda i: (i, 0)),
            pl.BlockSpec((BN_, D), lambda i: (i, 0)),
        ],
        out_shape=[
            jax.ShapeDtypeStruct((N, NSC), jnp.float32),
            jax.ShapeDtypeStruct((N, D), jnp.float32),
        ],
    )(node_feature, W_node, bn2, W_skip, bs2)

    xh = _sc_gather(h, dst2d, N, NSC)

    BE = 1000
    m1 = pl.pallas_call(
        _edge1_body,
        grid=(E // BE,),
        in_specs=[
            pl.BlockSpec((BE, 16), lambda i: (i, 0)),
            pl.BlockSpec((BE, 3), lambda i: (i, 0)),
            pl.BlockSpec((BE, NSC), lambda i: (i, 0)),
            pl.BlockSpec((16, 16), lambda i: (0, 0)),
            pl.BlockSpec((1, 16), lambda i: (0, 0)),
            pl.BlockSpec((16, 544), lambda i: (0, 0)),
        ],
        out_specs=pl.BlockSpec((BE, F1), lambda i: (i, 0)),
        out_shape=jax.ShapeDtypeStruct((E, F1), jnp.float32),
    )(edge_feature, edge_vec, xh, fc1_W1, b1a, p1)

    o1p, cntp = _sc_scatter(m1, src2d, N, F1, with_count=True)

    BF = 1000
    o1 = pl.pallas_call(
        _fin1_body,
        grid=(N // BF,),
        in_specs=[
            pl.BlockSpec((2, BF, F1), lambda i: (0, i, 0)),
            pl.BlockSpec((2, BF, 8), lambda i: (0, i, 0)),
            pl.BlockSpec((BF, NSC), lambda i: (i, 0)),
        ],
        out_specs=pl.BlockSpec((BF, F1), lambda i: (i, 0)),
        out_shape=jax.ShapeDtypeStruct((N, F1), jnp.float32),
    )(o1p, cntp, h)

    xo = _sc_gather(o1, dst2d, N, F1)

    m2 = pl.pallas_call(
        _edge2_body,
        grid=(E // BE,),
        in_specs=[
            pl.BlockSpec((BE, 16), lambda i: (i, 0)),
            pl.BlockSpec((BE, 3), lambda i: (i, 0)),
            pl.BlockSpec((BE, F1), lambda i: (i, 0)),
            pl.BlockSpec((16, 16), lambda i: (0, 0)),
            pl.BlockSpec((1, 16), lambda i: (0, 0)),
            pl.BlockSpec((32, 272), lambda i: (0, 0)),
        ],
        out_specs=pl.BlockSpec((BE, NSC), lambda i: (i, 0)),
        out_shape=jax.ShapeDtypeStruct((E, NSC), jnp.float32),
    )(edge_feature, edge_vec, xo, fc2_W1, b2a, p2)

    o2p = _sc_scatter(m2, src2d, N, NSC, with_count=False)

    BP = 1000
    out = pl.pallas_call(
        functools.partial(_epi_body, N),
        grid=(2, N // BP),
        in_specs=[
            pl.BlockSpec((2, BP, NSC), lambda p, i: (0, i, 0)),
            pl.BlockSpec((2, BP, 8), lambda p, i: (0, i, 0)),
            pl.BlockSpec((BP, D), lambda p, i: (i, 0)),
            pl.BlockSpec((1, NSC), lambda p, i: (0, 0)),
            pl.BlockSpec((1, NSC), lambda p, i: (0, 0)),
            pl.BlockSpec((NSC, D), lambda p, i: (0, 0)),
            pl.BlockSpec((1, D), lambda p, i: (0, 0)),
        ],
        out_specs=pl.BlockSpec((BP, D), lambda p, i: (i, 0)),
        out_shape=jax.ShapeDtypeStruct((N, D), jnp.float32),
        scratch_shapes=[pltpu.VMEM((2, NSC), jnp.float32)],
    )(o2p, cntp, skip, gamma, beta, W_nl2, bnl)

    return out


# speedup vs baseline: 2.5848x; 2.5848x over previous
import functools

import jax
import jax.numpy as jnp
from jax import lax
from jax.experimental import pallas as pl
from jax.experimental.pallas import tpu as pltpu
from jax.experimental.pallas import tpu_sc as plsc

NSC = 16
NVC = 8
F1 = NSC + 3 * NVC + 5 * NVC
C1 = float(NSC) ** -0.5
C2 = float(NSC + 2 * NVC) ** -0.5
S3 = 3.0 ** 0.5
S5 = 5.0 ** 0.5

NW = 32
CHUNK = 128


def _softplus(x):
    return jnp.maximum(x, 0.0) + jnp.log(1.0 + jnp.exp(-jnp.abs(x)))


def _sh_cols(ev):
    x = ev[0:1, :]
    y = ev[1:2, :]
    z = ev[2:3, :]
    inv = 1.0 / (jnp.sqrt(x * x + y * y + z * z) + 1e-12)
    xn = x * inv
    yn = y * inv
    zn = z * inv
    sh1 = [S3 * xn, S3 * yn, S3 * zn]
    sh2 = [
        S5 * S3 * xn * zn,
        S5 * S3 * xn * yn,
        S5 * (yn * yn - 0.5 * (xn * xn + zn * zn)),
        S5 * S3 * yn * zn,
        S5 * (S3 / 2.0) * (zn * zn - xn * xn),
    ]
    return sh1, sh2


def _node_body(nf, wn, bn, ws, bs, h_ref, skip_ref):
    x = nf[...]
    h_ref[...] = jnp.dot(x, wn[...], preferred_element_type=jnp.float32) + bn[...]
    skip_ref[...] = jnp.dot(x, ws[...], preferred_element_type=jnp.float32) + bs[...]


def _edge1_body(ef, ev, xh, w1at, b1c, p1t, m1_ref):
    sh1, sh2 = _sh_cols(ev[...].T)
    t = _softplus(
        jnp.dot(w1at[...], ef[...].T, preferred_element_type=jnp.float32)
        + b1c[...]
    )
    xt = xh[...].T
    a = jnp.dot(p1t[...], xt, preferred_element_type=jnp.float32)
    acc = a[512:544, :]
    for k in range(16):
        acc = acc + t[k : k + 1, :] * a[k * 32 : (k + 1) * 32, :]
    pre = acc * C1
    pre0 = pre[0:16, :]
    pre1 = pre[16:24, :]
    pre2 = pre[24:32, :]
    blocks = [pre0]
    for m in range(3):
        blocks.append(pre1 * sh1[m])
    for m in range(5):
        blocks.append(pre2 * sh2[m])
    m1_ref[...] = jnp.concatenate(blocks, axis=0).T


def _fin1_body(o1p, cntp, h, o1_ref):
    s = o1p[0] + o1p[1]
    c = cntp[0][:, 0:1] + cntp[1][:, 0:1]
    o1 = s / jnp.maximum(c, 1.0)
    o1_ref[...] = jnp.concatenate([o1[:, 0:16] + h[...], o1[:, 16:80]], axis=1)


def _edge2_body(ef, ev, xo, w2at, b2c, p2t, m2_ref):
    sh1, sh2 = _sh_cols(ev[...].T)
    t = _softplus(
        jnp.dot(w2at[...], ef[...].T, preferred_element_type=jnp.float32)
        + b2c[...]
    )
    f = xo[...].T
    r0 = f[0:16, :]
    r1 = (
        f[16:24, :] * sh1[0] + f[24:32, :] * sh1[1] + f[32:40, :] * sh1[2]
    ) * (3.0 ** -0.5)
    r2 = (
        f[40:48, :] * sh2[0]
        + f[48:56, :] * sh2[1]
        + f[56:64, :] * sh2[2]
        + f[64:72, :] * sh2[3]
        + f[72:80, :] * sh2[4]
    ) * (5.0 ** -0.5)
    r = jnp.concatenate([r0, r1, r2], axis=0)
    a = jnp.dot(p2t[...], r, preferred_element_type=jnp.float32)
    acc = a[256:272, :]
    for k in range(16):
        acc = acc + t[k : k + 1, :] * a[k * 16 : (k + 1) * 16, :]
    m2_ref[...] = (acc * C2).T


def _epi_body(n_nodes, o2p, cntp, skip, gamma, beta, wnl, bnl, out_ref, stat_ref):
    ph = pl.program_id(0)
    i = pl.program_id(1)
    s = o2p[0] + o2p[1]
    c = cntp[0][:, 0:1] + cntp[1][:, 0:1]
    o2 = s / jnp.maximum(c, 1.0)

    @pl.when(jnp.logical_and(ph == 0, i == 0))
    def _():
        stat_ref[...] = jnp.zeros_like(stat_ref)

    @pl.when(ph == 0)
    def _():
        stat_ref[...] = stat_ref[...] + jnp.concatenate(
            [
                jnp.sum(o2, axis=0, keepdims=True),
                jnp.sum(o2 * o2, axis=0, keepdims=True),
            ],
            axis=0,
        )

    @pl.when(ph == 1)
    def _():
        inv_n = 1.0 / float(n_nodes)
        mu = stat_ref[0:1, :] * inv_n
        var = stat_ref[1:2, :] * inv_n - mu * mu
        xb = (o2 - mu) / jnp.sqrt(var + 1e-5) * gamma[...] + beta[...]
        y = _softplus(
            jnp.dot(_softplus(xb), wnl[...], preferred_element_type=jnp.float32)
            + bnl[...]
        )
        out_ref[...] = y + skip[...]


def _sc_gather(table, idx2d, n_rows, width):
    R = idx2d.shape[0]
    rows_per = -(-R // NW)
    mesh = plsc.VectorSubcoreMesh(core_axis_name="c", subcore_axis_name="s")

    @functools.partial(
        pl.kernel,
        out_type=jax.ShapeDtypeStruct((R * CHUNK, width), jnp.float32),
        mesh=mesh,
        scratch_types=[
            pltpu.VMEM((CHUNK,), jnp.int32),
            pltpu.VMEM((CHUNK, width), jnp.float32),
            pltpu.SemaphoreType.DMA,
        ],
        compiler_params=pltpu.CompilerParams(use_tc_tiling_on_sc=False),
    )
    def gk(table_hbm, idx_hbm, out_hbm, idx_v, rows_v, sem):
        cid = lax.axis_index("c")
        sid = lax.axis_index("s")
        wid = sid * 2 + cid

        def body(j, carry):
            row = wid * rows_per + j

            @pl.when(row < R)
            def _():
                pltpu.sync_copy(idx_hbm.at[row], idx_v)
                pltpu.async_copy(table_hbm.at[idx_v], rows_v, sem).wait()
                pltpu.sync_copy(rows_v, out_hbm.at[pl.ds(row * CHUNK, CHUNK)])

            return carry

        lax.fori_loop(0, rows_per, body, 0)

    return gk(table, idx2d)


def _sc_scatter(msg, idx2d, n_nodes, width, with_count):
    R = idx2d.shape[0]
    rows_per = -(-R // NW)
    npsub = n_nodes // 16
    mesh = plsc.VectorSubcoreMesh(core_axis_name="c", subcore_axis_name="s")

    zacc = jnp.zeros((n_nodes, width), jnp.float32)
    out_type = [jax.ShapeDtypeStruct((2, n_nodes, width), jnp.float32)]
    scratch = [
        pltpu.VMEM((CHUNK,), jnp.int32),
        pltpu.VMEM((CHUNK, width), jnp.float32),
        pltpu.VMEM_SHARED((n_nodes, width), jnp.float32),
    ]
    if with_count:
        zcnt = jnp.zeros((n_nodes, 8), jnp.float32)
        ones = jnp.ones((CHUNK, 8), jnp.float32)
        out_type.append(jax.ShapeDtypeStruct((2, n_nodes, 8), jnp.float32))
        scratch += [
            pltpu.VMEM((CHUNK, 8), jnp.float32),
            pltpu.VMEM_SHARED((n_nodes, 8), jnp.float32),
        ]
    scratch.append(pltpu.SemaphoreType.DMA)

    @functools.partial(
        pl.kernel, out_type=tuple(out_type), mesh=mesh, scratch_types=scratch,
        compiler_params=pltpu.CompilerParams(use_tc_tiling_on_sc=False),
    )
    def sk(*refs):
        if with_count:
            (m_hbm, idx_hbm, za_hbm, zc_hbm, ones_hbm, accp_hbm, cntp_hbm,
             idx_v, rows_v, acc_sh, ones_v, cnt_sh, sem) = refs
        else:
            (m_hbm, idx_hbm, za_hbm, accp_hbm,
             idx_v, rows_v, acc_sh, sem) = refs
        cid = lax.axis_index("c")
        sid = lax.axis_index("s")
        wid = sid * 2 + cid

        pltpu.sync_copy(
            za_hbm.at[pl.ds(sid * npsub, npsub)],
            acc_sh.at[pl.ds(sid * npsub, npsub)],
        )
        if with_count:
            pltpu.sync_copy(
                zc_hbm.at[pl.ds(sid * npsub, npsub)],
                cnt_sh.at[pl.ds(sid * npsub, npsub)],
            )
            pltpu.sync_copy(ones_hbm, ones_v)
        plsc.subcore_barrier()

        def body(j, carry):
            row = wid * rows_per + j

            @pl.when(row < R)
            def _():
                pltpu.sync_copy(idx_hbm.at[row], idx_v)
                pltpu.sync_copy(m_hbm.at[pl.ds(row * CHUNK, CHUNK)], rows_v)
                pltpu.sync_copy(rows_v, acc_sh.at[idx_v], add=True)
                if with_count:
                    pltpu.sync_copy(ones_v, cnt_sh.at[idx_v], add=True)

            return carry

        lax.fori_loop(0, rows_per, body, 0)
        plsc.subcore_barrier()

        pltpu.sync_copy(
            acc_sh.at[pl.ds(sid * npsub, npsub)],
            accp_hbm.at[cid, pl.ds(sid * npsub, npsub)],
        )
        if with_count:
            pltpu.sync_copy(
                cnt_sh.at[pl.ds(sid * npsub, npsub)],
                cntp_hbm.at[cid, pl.ds(sid * npsub, npsub)],
            )

    if with_count:
        return sk(msg, idx2d, zacc, zcnt, ones)
    return sk(msg, idx2d, zacc)[0]


def _build_p1(fc1_W2, fc1_b2):
    w0 = fc1_W2[:, :256].reshape(16, 16, 16)
    w1 = fc1_W2[:, 256:384].reshape(16, 16, 8)
    w2 = fc1_W2[:, 384:512].reshape(16, 16, 8)
    p = jnp.concatenate([w0, w1, w2], axis=2)
    p = p.transpose(1, 0, 2).reshape(16, 512)
    b = jnp.concatenate(
        [
            fc1_b2[:256].reshape(16, 16),
            fc1_b2[256:384].reshape(16, 8),
            fc1_b2[384:512].reshape(16, 8),
        ],
        axis=1,
    )
    return jnp.concatenate([p, b], axis=1)


def _build_p2(fc2_W2, fc2_b2):
    w00 = fc2_W2[:, :256].reshape(16, 16, 16)
    w11 = fc2_W2[:, 256:384].reshape(16, 8, 16)
    w22 = fc2_W2[:, 384:512].reshape(16, 8, 16)
    p = jnp.concatenate([w00, w11, w22], axis=1)
    p = p.transpose(1, 0, 2).reshape(32, 256)
    b = jnp.concatenate(
        [
            fc2_b2[:256].reshape(16, 16),
            fc2_b2[256:384].reshape(8, 16),
            fc2_b2[384:512].reshape(8, 16),
        ],
        axis=0,
    )
    return jnp.concatenate([p, b], axis=1)


def kernel(node_feature, edge_index, edge_feature, edge_vec, W_node, b_node,
           W_skip, b_skip, fc1_W1, fc1_b1, fc1_W2, fc1_b2, fc2_W1, fc2_b1,
           fc2_W2, fc2_b2, bn_gamma, bn_beta, W_nl2, b_nl2):
    N, D = node_feature.shape
    E = edge_index.shape[1]
    R = E // CHUNK

    src2d = edge_index[0].reshape(R, CHUNK).astype(jnp.int32)
    dst2d = edge_index[1].reshape(R, CHUNK).astype(jnp.int32)

    p1t = _build_p1(fc1_W2, fc1_b2).T
    p2t = _build_p2(fc2_W2, fc2_b2).T
    w1at = fc1_W1.T
    w2at = fc2_W1.T
    bn2 = b_node.reshape(1, NSC)
    bs2 = b_skip.reshape(1, D)
    b1c = fc1_b1.reshape(16, 1)
    b2c = fc2_b1.reshape(16, 1)
    gamma = bn_gamma.reshape(1, NSC)
    beta = bn_beta.reshape(1, NSC)
    bnl = b_nl2.reshape(1, D)

    BN_ = 1000
    h, skip = pl.pallas_call(
        _node_body,
        grid=(N // BN_,),
        in_specs=[
            pl.BlockSpec((BN_, D), lambda i: (i, 0)),
            pl.BlockSpec((D, NSC), lambda i: (0, 0)),
            pl.BlockSpec((1, NSC), lambda i: (0, 0)),
            pl.BlockSpec((D, D), lambda i: (0, 0)),
            pl.BlockSpec((1, D), lambda i: (0, 0)),
        ],
        out_specs=[
            pl.BlockSpec((BN_, NSC), lambda i: (i, 0)),
            pl.BlockSpec((BN_, D), lambda i: (i, 0)),
        ],
        out_shape=[
            jax.ShapeDtypeStruct((N, NSC), jnp.float32),
            jax.ShapeDtypeStruct((N, D), jnp.float32),
        ],
    )(node_feature, W_node, bn2, W_skip, bs2)

    xh = _sc_gather(h, dst2d, N, NSC)

    BE = 640
    m1 = pl.pallas_call(
        _edge1_body,
        grid=(E // BE,),
        in_specs=[
            pl.BlockSpec((BE, 16), lambda i: (i, 0)),
            pl.BlockSpec((BE, 3), lambda i: (i, 0)),
            pl.BlockSpec((BE, NSC), lambda i: (i, 0)),
            pl.BlockSpec((16, 16), lambda i: (0, 0)),
            pl.BlockSpec((16, 1), lambda i: (0, 0)),
            pl.BlockSpec((544, 16), lambda i: (0, 0)),
        ],
        out_specs=pl.BlockSpec((BE, F1), lambda i: (i, 0)),
        out_shape=jax.ShapeDtypeStruct((E, F1), jnp.float32),
    )(edge_feature, edge_vec, xh, w1at, b1c, p1t)

    o1p, cntp = _sc_scatter(m1, src2d, N, F1, with_count=True)

    BF = 1000
    o1 = pl.pallas_call(
        _fin1_body,
        grid=(N // BF,),
        in_specs=[
            pl.BlockSpec((2, BF, F1), lambda i: (0, i, 0)),
            pl.BlockSpec((2, BF, 8), lambda i: (0, i, 0)),
            pl.BlockSpec((BF, NSC), lambda i: (i, 0)),
        ],
        out_specs=pl.BlockSpec((BF, F1), lambda i: (i, 0)),
        out_shape=jax.ShapeDtypeStruct((N, F1), jnp.float32),
    )(o1p, cntp, h)

    xo = _sc_gather(o1, dst2d, N, F1)

    m2 = pl.pallas_call(
        _edge2_body,
        grid=(E // BE,),
        in_specs=[
            pl.BlockSpec((BE, 16), lambda i: (i, 0)),
            pl.BlockSpec((BE, 3), lambda i: (i, 0)),
            pl.BlockSpec((BE, F1), lambda i: (i, 0)),
            pl.BlockSpec((16, 16), lambda i: (0, 0)),
            pl.BlockSpec((16, 1), lambda i: (0, 0)),
            pl.BlockSpec((272, 32), lambda i: (0, 0)),
        ],
        out_specs=pl.BlockSpec((BE, NSC), lambda i: (i, 0)),
        out_shape=jax.ShapeDtypeStruct((E, NSC), jnp.float32),
    )(edge_feature, edge_vec, xo, w2at, b2c, p2t)

    o2p = _sc_scatter(m2, src2d, N, NSC, with_count=False)

    BP = 1000
    out = pl.pallas_call(
        functools.partial(_epi_body, N),
        grid=(2, N // BP),
        in_specs=[
            pl.BlockSpec((2, BP, NSC), lambda p, i: (0, i, 0)),
            pl.BlockSpec((2, BP, 8), lambda p, i: (0, i, 0)),
            pl.BlockSpec((BP, D), lambda p, i: (i, 0)),
            pl.BlockSpec((1, NSC), lambda p, i: (0, 0)),
            pl.BlockSpec((1, NSC), lambda p, i: (0, 0)),
            pl.BlockSpec((NSC, D), lambda p, i: (0, 0)),
            pl.BlockSpec((1, D), lambda p, i: (0, 0)),
        ],
        out_specs=pl.BlockSpec((BP, D), lambda p, i: (i, 0)),
        out_shape=jax.ShapeDtypeStruct((N, D), jnp.float32),
        scratch_shapes=[pltpu.VMEM((2, NSC), jnp.float32)],
    )(o2p, cntp, skip, gamma, beta, W_nl2, bnl)

    return out


# speedup vs baseline: 2.9084x; 1.1252x over previous
import functools

import jax
import jax.numpy as jnp
from jax import lax
from jax.experimental import pallas as pl
from jax.experimental.pallas import tpu as pltpu
from jax.experimental.pallas import tpu_sc as plsc

NSC = 16
NVC = 8
F1 = NSC + 3 * NVC + 5 * NVC
C1 = float(NSC) ** -0.5
C2 = float(NSC + 2 * NVC) ** -0.5
S3 = 3.0 ** 0.5
S5 = 5.0 ** 0.5

NW = 32
CHUNK = 128


def _softplus(x):
    return jnp.maximum(x, 0.0) + jnp.log(1.0 + jnp.exp(-jnp.abs(x)))


def _sh_cols(ev):
    x = ev[0:1, :]
    y = ev[1:2, :]
    z = ev[2:3, :]
    inv = 1.0 / (jnp.sqrt(x * x + y * y + z * z) + 1e-12)
    xn = x * inv
    yn = y * inv
    zn = z * inv
    sh1 = [S3 * xn, S3 * yn, S3 * zn]
    sh2 = [
        S5 * S3 * xn * zn,
        S5 * S3 * xn * yn,
        S5 * (yn * yn - 0.5 * (xn * xn + zn * zn)),
        S5 * S3 * yn * zn,
        S5 * (S3 / 2.0) * (zn * zn - xn * xn),
    ]
    return sh1, sh2


def _node_body(nf, wn, bn, ws, bs, h_ref, skip_ref):
    x = nf[...]
    h_ref[...] = jnp.dot(x, wn[...], preferred_element_type=jnp.float32) + bn[...]
    skip_ref[...] = jnp.dot(x, ws[...], preferred_element_type=jnp.float32) + bs[...]


def _edge1_body(ef, ev, xh, w1at, b1c, p1t, m1_ref):
    sh1, sh2 = _sh_cols(ev[...].T)
    t = _softplus(
        jnp.dot(w1at[...], ef[...].T, preferred_element_type=jnp.float32)
        + b1c[...]
    )
    xt = xh[...].T
    a = jnp.dot(p1t[...], xt, preferred_element_type=jnp.float32)
    acc = a[512:544, :]
    for k in range(16):
        acc = acc + t[k : k + 1, :] * a[k * 32 : (k + 1) * 32, :]
    pre = acc * C1
    pre0 = pre[0:16, :]
    pre1 = pre[16:24, :]
    pre2 = pre[24:32, :]
    blocks = [pre0]
    for m in range(3):
        blocks.append(pre1 * sh1[m])
    for m in range(5):
        blocks.append(pre2 * sh2[m])
    m1_ref[...] = jnp.concatenate(blocks, axis=0).T


def _fin1_body(o1p, cntp, h, o1_ref):
    s = o1p[0] + o1p[1]
    c = cntp[0][:, 0:1] + cntp[1][:, 0:1]
    o1 = s / jnp.maximum(c, 1.0)
    o1_ref[...] = jnp.concatenate([o1[:, 0:16] + h[...], o1[:, 16:80]], axis=1)


def _edge2_body(ef, ev, xo, w2at, b2c, p2t, m2_ref):
    sh1, sh2 = _sh_cols(ev[...].T)
    t = _softplus(
        jnp.dot(w2at[...], ef[...].T, preferred_element_type=jnp.float32)
        + b2c[...]
    )
    f = xo[...].T
    r0 = f[0:16, :]
    r1 = (
        f[16:24, :] * sh1[0] + f[24:32, :] * sh1[1] + f[32:40, :] * sh1[2]
    ) * (3.0 ** -0.5)
    r2 = (
        f[40:48, :] * sh2[0]
        + f[48:56, :] * sh2[1]
        + f[56:64, :] * sh2[2]
        + f[64:72, :] * sh2[3]
        + f[72:80, :] * sh2[4]
    ) * (5.0 ** -0.5)
    r = jnp.concatenate([r0, r1, r2], axis=0)
    a = jnp.dot(p2t[...], r, preferred_element_type=jnp.float32)
    acc = a[256:272, :]
    for k in range(16):
        acc = acc + t[k : k + 1, :] * a[k * 16 : (k + 1) * 16, :]
    m2_ref[...] = (acc * C2).T


def _epi_body(n_nodes, o2p, cntp, skip, gamma, beta, wnl, bnl, out_ref, stat_ref):
    ph = pl.program_id(0)
    i = pl.program_id(1)
    s = o2p[0] + o2p[1]
    c = cntp[0][:, 0:1] + cntp[1][:, 0:1]
    o2 = s / jnp.maximum(c, 1.0)

    @pl.when(jnp.logical_and(ph == 0, i == 0))
    def _():
        stat_ref[...] = jnp.zeros_like(stat_ref)

    @pl.when(ph == 0)
    def _():
        stat_ref[...] = stat_ref[...] + jnp.concatenate(
            [
                jnp.sum(o2, axis=0, keepdims=True),
                jnp.sum(o2 * o2, axis=0, keepdims=True),
            ],
            axis=0,
        )

    @pl.when(ph == 1)
    def _():
        inv_n = 1.0 / float(n_nodes)
        mu = stat_ref[0:1, :] * inv_n
        var = stat_ref[1:2, :] * inv_n - mu * mu
        xb = (o2 - mu) / jnp.sqrt(var + 1e-5) * gamma[...] + beta[...]
        y = _softplus(
            jnp.dot(_softplus(xb), wnl[...], preferred_element_type=jnp.float32)
            + bnl[...]
        )
        out_ref[...] = y + skip[...]


GRP = 4


def _sc_gather(table, idxp, R, width, grp=GRP):
    rows_per = idxp.shape[0] // NW
    npairs = rows_per // (2 * grp)
    mesh = plsc.VectorSubcoreMesh(core_axis_name="c", subcore_axis_name="s")

    @functools.partial(
        pl.kernel,
        out_type=jax.ShapeDtypeStruct((R * CHUNK, width), jnp.float32),
        mesh=mesh,
        scratch_types=[
            pltpu.VMEM((rows_per, CHUNK), jnp.int32),
            pltpu.VMEM((2 * grp, CHUNK, width), jnp.float32),
            pltpu.SemaphoreType.DMA,
            pltpu.SemaphoreType.DMA,
            pltpu.SemaphoreType.DMA,
            pltpu.SemaphoreType.DMA,
        ],
        compiler_params=pltpu.CompilerParams(use_tc_tiling_on_sc=False),
    )
    def gk(table_hbm, idx_hbm, out_hbm, idx_all, bufs, sg0, sg1, ss0, ss1):
        cid = lax.axis_index("c")
        sid = lax.axis_index("s")
        wid = sid * 2 + cid
        base = wid * rows_per
        pltpu.sync_copy(idx_hbm.at[pl.ds(base, rows_per)], idx_all)
        sgs = (sg0, sg1)
        sss = (ss0, ss1)

        def gather_desc(g, q, s):
            j = g * grp + q
            return pltpu.make_async_copy(
                table_hbm.at[idx_all.at[j]], bufs.at[s * grp + q], sgs[s]
            )

        def store_desc(g, q, s):
            j = g * grp + q
            return pltpu.make_async_copy(
                bufs.at[s * grp + q],
                out_hbm.at[pl.ds((base + j) * CHUNK, CHUNK)],
                sss[s],
            )

        def live(g, q):
            return base + g * grp + q < R

        def body(jp, carry):
            g0 = jp * 2
            g1 = g0 + 1
            for q in range(grp):
                @pl.when(jnp.logical_and(g0 >= 2, live(g0 - 2, q)))
                def _(q=q):
                    store_desc(g0 - 2, q, 0).wait()
            for q in range(grp):
                @pl.when(live(g0, q))
                def _(q=q):
                    gather_desc(g0, q, 0).start()
            for q in range(grp):
                @pl.when(jnp.logical_and(g0 >= 2, live(g0 - 1, q)))
                def _(q=q):
                    store_desc(g0 - 1, q, 1).wait()
            for q in range(grp):
                @pl.when(live(g1, q))
                def _(q=q):
                    gather_desc(g1, q, 1).start()
            for q in range(grp):
                @pl.when(live(g0, q))
                def _(q=q):
                    gather_desc(g0, q, 0).wait()
                    store_desc(g0, q, 0).start()
            for q in range(grp):
                @pl.when(live(g1, q))
                def _(q=q):
                    gather_desc(g1, q, 1).wait()
                    store_desc(g1, q, 1).start()
            return carry

        lax.fori_loop(0, npairs, body, 0)
        gl0 = (npairs - 1) * 2
        for q in range(grp):
            @pl.when(live(gl0, q))
            def _(q=q):
                store_desc(gl0, q, 0).wait()
        for q in range(grp):
            @pl.when(live(gl0 + 1, q))
            def _(q=q):
                store_desc(gl0 + 1, q, 1).wait()

    return gk(table, idxp)


def _sc_scatter(msg, idxp, R, n_nodes, width, with_count, grp=GRP):
    rows_per = idxp.shape[0] // NW
    npairs = rows_per // (2 * grp)
    npsub = n_nodes // 16
    mesh = plsc.VectorSubcoreMesh(core_axis_name="c", subcore_axis_name="s")

    zacc = jnp.zeros((n_nodes, width), jnp.float32)
    out_type = [jax.ShapeDtypeStruct((2, n_nodes, width), jnp.float32)]
    scratch = [
        pltpu.VMEM((rows_per, CHUNK), jnp.int32),
        pltpu.VMEM((2 * grp, CHUNK, width), jnp.float32),
        pltpu.VMEM_SHARED((n_nodes, width), jnp.float32),
    ]
    if with_count:
        zcnt = jnp.zeros((n_nodes, 8), jnp.float32)
        ones = jnp.ones((CHUNK, 8), jnp.float32)
        out_type.append(jax.ShapeDtypeStruct((2, n_nodes, 8), jnp.float32))
        scratch += [
            pltpu.VMEM((CHUNK, 8), jnp.float32),
            pltpu.VMEM_SHARED((n_nodes, 8), jnp.float32),
        ]
    scratch += [pltpu.SemaphoreType.DMA] * 6

    @functools.partial(
        pl.kernel, out_type=tuple(out_type), mesh=mesh, scratch_types=scratch,
        compiler_params=pltpu.CompilerParams(use_tc_tiling_on_sc=False),
    )
    def sk(*refs):
        if with_count:
            (m_hbm, idx_hbm, za_hbm, zc_hbm, ones_hbm, accp_hbm, cntp_hbm,
             idx_all, bufs, acc_sh, ones_v, cnt_sh,
             sl0, sl1, sa0, sa1, sc0, sc1) = refs
        else:
            (m_hbm, idx_hbm, za_hbm, accp_hbm,
             idx_all, bufs, acc_sh, sl0, sl1, sa0, sa1, sc0, sc1) = refs
        cid = lax.axis_index("c")
        sid = lax.axis_index("s")
        wid = sid * 2 + cid
        base = wid * rows_per

        pltpu.sync_copy(
            za_hbm.at[pl.ds(sid * npsub, npsub)],
            acc_sh.at[pl.ds(sid * npsub, npsub)],
        )
        if with_count:
            pltpu.sync_copy(
                zc_hbm.at[pl.ds(sid * npsub, npsub)],
                cnt_sh.at[pl.ds(sid * npsub, npsub)],
            )
            pltpu.sync_copy(ones_hbm, ones_v)
        pltpu.sync_copy(idx_hbm.at[pl.ds(base, rows_per)], idx_all)
        plsc.subcore_barrier()

        sls = (sl0, sl1)
        sas = (sa0, sa1)
        scs = (sc0, sc1)

        def load_desc(g, q, s):
            j = g * grp + q
            return pltpu.make_async_copy(
                m_hbm.at[pl.ds((base + j) * CHUNK, CHUNK)],
                bufs.at[s * grp + q],
                sls[s],
            )

        def add_desc(g, q, s):
            j = g * grp + q
            return pltpu.make_async_copy(
                bufs.at[s * grp + q], acc_sh.at[idx_all.at[j]], sas[s]
            )

        def cnt_desc(g, q, s):
            j = g * grp + q
            return pltpu.make_async_copy(
                ones_v, cnt_sh.at[idx_all.at[j]], scs[s]
            )

        def live(g, q):
            return base + g * grp + q < R

        def body(jp, carry):
            g0 = jp * 2
            g1 = g0 + 1
            for q in range(grp):
                @pl.when(jnp.logical_and(g0 >= 2, live(g0 - 2, q)))
                def _(q=q):
                    add_desc(g0 - 2, q, 0).wait()
                    if with_count:
                        cnt_desc(g0 - 2, q, 0).wait()
            for q in range(grp):
                @pl.when(live(g0, q))
                def _(q=q):
                    load_desc(g0, q, 0).start()
            for q in range(grp):
                @pl.when(jnp.logical_and(g0 >= 2, live(g0 - 1, q)))
                def _(q=q):
                    add_desc(g0 - 1, q, 1).wait()
                    if with_count:
                        cnt_desc(g0 - 1, q, 1).wait()
            for q in range(grp):
                @pl.when(live(g1, q))
                def _(q=q):
                    load_desc(g1, q, 1).start()
            for q in range(grp):
                @pl.when(live(g0, q))
                def _(q=q):
                    load_desc(g0, q, 0).wait()
                    add_desc(g0, q, 0).start(add=True)
                    if with_count:
                        cnt_desc(g0, q, 0).start(add=True)
            for q in range(grp):
                @pl.when(live(g1, q))
                def _(q=q):
                    load_desc(g1, q, 1).wait()
                    add_desc(g1, q, 1).start(add=True)
                    if with_count:
                        cnt_desc(g1, q, 1).start(add=True)
            return carry

        lax.fori_loop(0, npairs, body, 0)
        gl0 = (npairs - 1) * 2
        for q in range(grp):
            @pl.when(live(gl0, q))
            def _(q=q):
                add_desc(gl0, q, 0).wait()
                if with_count:
                    cnt_desc(gl0, q, 0).wait()
        for q in range(grp):
            @pl.when(live(gl0 + 1, q))
            def _(q=q):
                add_desc(gl0 + 1, q, 1).wait()
                if with_count:
                    cnt_desc(gl0 + 1, q, 1).wait()
        plsc.subcore_barrier()

        pltpu.sync_copy(
            acc_sh.at[pl.ds(sid * npsub, npsub)],
            accp_hbm.at[cid, pl.ds(sid * npsub, npsub)],
        )
        if with_count:
            pltpu.sync_copy(
                cnt_sh.at[pl.ds(sid * npsub, npsub)],
                cntp_hbm.at[cid, pl.ds(sid * npsub, npsub)],
            )

    if with_count:
        return sk(msg, idxp, zacc, zcnt, ones)
    return sk(msg, idxp, zacc)[0]


def _build_p1(fc1_W2, fc1_b2):
    w0 = fc1_W2[:, :256].reshape(16, 16, 16)
    w1 = fc1_W2[:, 256:384].reshape(16, 16, 8)
    w2 = fc1_W2[:, 384:512].reshape(16, 16, 8)
    p = jnp.concatenate([w0, w1, w2], axis=2)
    p = p.transpose(1, 0, 2).reshape(16, 512)
    b = jnp.concatenate(
        [
            fc1_b2[:256].reshape(16, 16),
            fc1_b2[256:384].reshape(16, 8),
            fc1_b2[384:512].reshape(16, 8),
        ],
        axis=1,
    )
    return jnp.concatenate([p, b], axis=1)


def _build_p2(fc2_W2, fc2_b2):
    w00 = fc2_W2[:, :256].reshape(16, 16, 16)
    w11 = fc2_W2[:, 256:384].reshape(16, 8, 16)
    w22 = fc2_W2[:, 384:512].reshape(16, 8, 16)
    p = jnp.concatenate([w00, w11, w22], axis=1)
    p = p.transpose(1, 0, 2).reshape(32, 256)
    b = jnp.concatenate(
        [
            fc2_b2[:256].reshape(16, 16),
            fc2_b2[256:384].reshape(8, 16),
            fc2_b2[384:512].reshape(8, 16),
        ],
        axis=0,
    )
    return jnp.concatenate([p, b], axis=1)


def kernel(node_feature, edge_index, edge_feature, edge_vec, W_node, b_node,
           W_skip, b_skip, fc1_W1, fc1_b1, fc1_W2, fc1_b2, fc2_W1, fc2_b1,
           fc2_W2, fc2_b2, bn_gamma, bn_beta, W_nl2, b_nl2):
    N, D = node_feature.shape
    E = edge_index.shape[1]
    R = E // CHUNK

    rows_per = -(-R // NW)
    rows_per += (-rows_per) % (2 * GRP)
    pad = NW * rows_per - R
    src2d = jnp.pad(edge_index[0].reshape(R, CHUNK).astype(jnp.int32),
                    ((0, pad), (0, 0)))
    dst2d = jnp.pad(edge_index[1].reshape(R, CHUNK).astype(jnp.int32),
                    ((0, pad), (0, 0)))

    p1t = _build_p1(fc1_W2, fc1_b2).T
    p2t = _build_p2(fc2_W2, fc2_b2).T
    w1at = fc1_W1.T
    w2at = fc2_W1.T
    bn2 = b_node.reshape(1, NSC)
    bs2 = b_skip.reshape(1, D)
    b1c = fc1_b1.reshape(16, 1)
    b2c = fc2_b1.reshape(16, 1)
    gamma = bn_gamma.reshape(1, NSC)
    beta = bn_beta.reshape(1, NSC)
    bnl = b_nl2.reshape(1, D)

    BN_ = 1000
    h, skip = pl.pallas_call(
        _node_body,
        grid=(N // BN_,),
        in_specs=[
            pl.BlockSpec((BN_, D), lambda i: (i, 0)),
            pl.BlockSpec((D, NSC), lambda i: (0, 0)),
            pl.BlockSpec((1, NSC), lambda i: (0, 0)),
            pl.BlockSpec((D, D), lambda i: (0, 0)),
            pl.BlockSpec((1, D), lambda i: (0, 0)),
        ],
        out_specs=[
            pl.BlockSpec((BN_, NSC), lambda i: (i, 0)),
            pl.BlockSpec((BN_, D), lambda i: (i, 0)),
        ],
        out_shape=[
            jax.ShapeDtypeStruct((N, NSC), jnp.float32),
            jax.ShapeDtypeStruct((N, D), jnp.float32),
        ],
    )(node_feature, W_node, bn2, W_skip, bs2)

    xh = _sc_gather(h, dst2d, R, NSC)

    BE = 640
    m1 = pl.pallas_call(
        _edge1_body,
        grid=(E // BE,),
        in_specs=[
            pl.BlockSpec((BE, 16), lambda i: (i, 0)),
            pl.BlockSpec((BE, 3), lambda i: (i, 0)),
            pl.BlockSpec((BE, NSC), lambda i: (i, 0)),
            pl.BlockSpec((16, 16), lambda i: (0, 0)),
            pl.BlockSpec((16, 1), lambda i: (0, 0)),
            pl.BlockSpec((544, 16), lambda i: (0, 0)),
        ],
        out_specs=pl.BlockSpec((BE, F1), lambda i: (i, 0)),
        out_shape=jax.ShapeDtypeStruct((E, F1), jnp.float32),
    )(edge_feature, edge_vec, xh, w1at, b1c, p1t)

    o1p, cntp = _sc_scatter(m1, src2d, R, N, F1, with_count=True, grp=2)

    BF = 1000
    o1 = pl.pallas_call(
        _fin1_body,
        grid=(N // BF,),
        in_specs=[
            pl.BlockSpec((2, BF, F1), lambda i: (0, i, 0)),
            pl.BlockSpec((2, BF, 8), lambda i: (0, i, 0)),
            pl.BlockSpec((BF, NSC), lambda i: (i, 0)),
        ],
        out_specs=pl.BlockSpec((BF, F1), lambda i: (i, 0)),
        out_shape=jax.ShapeDtypeStruct((N, F1), jnp.float32),
    )(o1p, cntp, h)

    xo = _sc_gather(o1, dst2d, R, F1)

    m2 = pl.pallas_call(
        _edge2_body,
        grid=(E // BE,),
        in_specs=[
            pl.BlockSpec((BE, 16), lambda i: (i, 0)),
            pl.BlockSpec((BE, 3), lambda i: (i, 0)),
            pl.BlockSpec((BE, F1), lambda i: (i, 0)),
            pl.BlockSpec((16, 16), lambda i: (0, 0)),
            pl.BlockSpec((16, 1), lambda i: (0, 0)),
            pl.BlockSpec((272, 32), lambda i: (0, 0)),
        ],
        out_specs=pl.BlockSpec((BE, NSC), lambda i: (i, 0)),
        out_shape=jax.ShapeDtypeStruct((E, NSC), jnp.float32),
    )(edge_feature, edge_vec, xo, w2at, b2c, p2t)

    o2p = _sc_scatter(m2, src2d, R, N, NSC, with_count=False)

    BP = 1000
    out = pl.pallas_call(
        functools.partial(_epi_body, N),
        grid=(2, N // BP),
        in_specs=[
            pl.BlockSpec((2, BP, NSC), lambda p, i: (0, i, 0)),
            pl.BlockSpec((2, BP, 8), lambda p, i: (0, i, 0)),
            pl.BlockSpec((BP, D), lambda p, i: (i, 0)),
            pl.BlockSpec((1, NSC), lambda p, i: (0, 0)),
            pl.BlockSpec((1, NSC), lambda p, i: (0, 0)),
            pl.BlockSpec((NSC, D), lambda p, i: (0, 0)),
            pl.BlockSpec((1, D), lambda p, i: (0, 0)),
        ],
        out_specs=pl.BlockSpec((BP, D), lambda p, i: (i, 0)),
        out_shape=jax.ShapeDtypeStruct((N, D), jnp.float32),
        scratch_shapes=[pltpu.VMEM((2, NSC), jnp.float32)],
    )(o2p, cntp, skip, gamma, beta, W_nl2, bnl)

    return out


# speedup vs baseline: 3.8318x; 1.3175x over previous
import functools

import jax
import jax.numpy as jnp
from jax import lax
from jax.experimental import pallas as pl
from jax.experimental.pallas import tpu as pltpu
from jax.experimental.pallas import tpu_sc as plsc

NSC = 16
NVC = 8
F1 = NSC + 3 * NVC + 5 * NVC
C1 = float(NSC) ** -0.5
C2 = float(NSC + 2 * NVC) ** -0.5
S3 = 3.0 ** 0.5
S5 = 5.0 ** 0.5

NW = 32
CHUNK = 128


def _softplus(x):
    return jnp.maximum(x, 0.0) + jnp.log(1.0 + jnp.exp(-jnp.abs(x)))


def _sh_cols(ev):
    x = ev[0:1, :]
    y = ev[1:2, :]
    z = ev[2:3, :]
    inv = 1.0 / (jnp.sqrt(x * x + y * y + z * z) + 1e-12)
    xn = x * inv
    yn = y * inv
    zn = z * inv
    sh1 = [S3 * xn, S3 * yn, S3 * zn]
    sh2 = [
        S5 * S3 * xn * zn,
        S5 * S3 * xn * yn,
        S5 * (yn * yn - 0.5 * (xn * xn + zn * zn)),
        S5 * S3 * yn * zn,
        S5 * (S3 / 2.0) * (zn * zn - xn * xn),
    ]
    return sh1, sh2


def _node_body(nf, wn, bn, ws, bs, h_ref, skip_ref):
    x = nf[...]
    h_ref[...] = jnp.dot(x, wn[...], preferred_element_type=jnp.float32) + bn[...]
    skip_ref[...] = jnp.dot(x, ws[...], preferred_element_type=jnp.float32) + bs[...]


def _edge1_body(ef, ev, xh, w1at, b1c, p1t, m1_ref):
    sh1, sh2 = _sh_cols(ev[...].T)
    t = _softplus(
        jnp.dot(w1at[...], ef[...].T, preferred_element_type=jnp.float32)
        + b1c[...]
    )
    xt = xh[...].T
    a = jnp.dot(p1t[...], xt, preferred_element_type=jnp.float32)
    acc = a[512:544, :]
    for k in range(16):
        acc = acc + t[k : k + 1, :] * a[k * 32 : (k + 1) * 32, :]
    pre = acc * C1
    pre0 = pre[0:16, :]
    pre1 = pre[16:24, :]
    pre2 = pre[24:32, :]
    blocks = [pre0]
    for m in range(3):
        blocks.append(pre1 * sh1[m])
    for m in range(5):
        blocks.append(pre2 * sh2[m])
    m1_ref[...] = jnp.concatenate(blocks, axis=0).T


def _fin1_body(o1p, cntp, h, o1_ref):
    s = o1p[0] + o1p[1]
    c = cntp[0][:, 0:1] + cntp[1][:, 0:1]
    o1 = s / jnp.maximum(c, 1.0)
    o1_ref[...] = jnp.concatenate([o1[:, 0:16] + h[...], o1[:, 16:80]], axis=1)


def _edge2_body(ef, ev, xo, w2at, b2c, p2t, m2_ref):
    sh1, sh2 = _sh_cols(ev[...].T)
    t = _softplus(
        jnp.dot(w2at[...], ef[...].T, preferred_element_type=jnp.float32)
        + b2c[...]
    )
    f = xo[...].T
    r0 = f[0:16, :]
    r1 = (
        f[16:24, :] * sh1[0] + f[24:32, :] * sh1[1] + f[32:40, :] * sh1[2]
    ) * (3.0 ** -0.5)
    r2 = (
        f[40:48, :] * sh2[0]
        + f[48:56, :] * sh2[1]
        + f[56:64, :] * sh2[2]
        + f[64:72, :] * sh2[3]
        + f[72:80, :] * sh2[4]
    ) * (5.0 ** -0.5)
    r = jnp.concatenate([r0, r1, r2], axis=0)
    a = jnp.dot(p2t[...], r, preferred_element_type=jnp.float32)
    acc = a[256:272, :]
    for k in range(16):
        acc = acc + t[k : k + 1, :] * a[k * 16 : (k + 1) * 16, :]
    m2_ref[...] = (acc * C2).T


def _epi_body(n_nodes, o2p, cntp, skip, gamma, beta, wnl, bnl, out_ref, stat_ref):
    ph = pl.program_id(0)
    i = pl.program_id(1)
    s = o2p[0] + o2p[1]
    c = cntp[0][:, 0:1] + cntp[1][:, 0:1]
    o2 = s / jnp.maximum(c, 1.0)

    @pl.when(jnp.logical_and(ph == 0, i == 0))
    def _():
        stat_ref[...] = jnp.zeros_like(stat_ref)

    @pl.when(ph == 0)
    def _():
        stat_ref[...] = stat_ref[...] + jnp.concatenate(
            [
                jnp.sum(o2, axis=0, keepdims=True),
                jnp.sum(o2 * o2, axis=0, keepdims=True),
            ],
            axis=0,
        )

    @pl.when(ph == 1)
    def _():
        inv_n = 1.0 / float(n_nodes)
        mu = stat_ref[0:1, :] * inv_n
        var = stat_ref[1:2, :] * inv_n - mu * mu
        xb = (o2 - mu) / jnp.sqrt(var + 1e-5) * gamma[...] + beta[...]
        y = _softplus(
            jnp.dot(_softplus(xb), wnl[...], preferred_element_type=jnp.float32)
            + bnl[...]
        )
        out_ref[...] = y + skip[...]


GRP = 4


def _sc_gather(table, idxp, R, width, grp=GRP):
    rows_per = idxp.shape[0] // NW
    npairs = rows_per // (2 * grp)
    mesh = plsc.VectorSubcoreMesh(core_axis_name="c", subcore_axis_name="s")

    @functools.partial(
        pl.kernel,
        out_type=jax.ShapeDtypeStruct((R * CHUNK, width), jnp.float32),
        mesh=mesh,
        scratch_types=[
            pltpu.VMEM((rows_per, CHUNK), jnp.int32),
            pltpu.VMEM((2 * grp, CHUNK, width), jnp.float32),
            pltpu.SemaphoreType.DMA,
            pltpu.SemaphoreType.DMA,
            pltpu.SemaphoreType.DMA,
            pltpu.SemaphoreType.DMA,
        ],
        compiler_params=pltpu.CompilerParams(use_tc_tiling_on_sc=False),
    )
    def gk(table_hbm, idx_hbm, out_hbm, idx_all, bufs, sg0, sg1, ss0, ss1):
        cid = lax.axis_index("c")
        sid = lax.axis_index("s")
        wid = sid * 2 + cid
        base = wid * rows_per
        pltpu.sync_copy(idx_hbm.at[pl.ds(base, rows_per)], idx_all)
        sgs = (sg0, sg1)
        sss = (ss0, ss1)

        def gather_desc(g, q, s):
            j = g * grp + q
            return pltpu.make_async_copy(
                table_hbm.at[idx_all.at[j]], bufs.at[s * grp + q], sgs[s]
            )

        def store_desc(g, q, s):
            j = g * grp + q
            return pltpu.make_async_copy(
                bufs.at[s * grp + q],
                out_hbm.at[pl.ds((base + j) * CHUNK, CHUNK)],
                sss[s],
            )

        def live(g, q):
            return base + g * grp + q < R

        def body(jp, carry):
            g0 = jp * 2
            g1 = g0 + 1
            for q in range(grp):
                @pl.when(jnp.logical_and(g0 >= 2, live(g0 - 2, q)))
                def _(q=q):
                    store_desc(g0 - 2, q, 0).wait()
            for q in range(grp):
                @pl.when(live(g0, q))
                def _(q=q):
                    gather_desc(g0, q, 0).start()
            for q in range(grp):
                @pl.when(jnp.logical_and(g0 >= 2, live(g0 - 1, q)))
                def _(q=q):
                    store_desc(g0 - 1, q, 1).wait()
            for q in range(grp):
                @pl.when(live(g1, q))
                def _(q=q):
                    gather_desc(g1, q, 1).start()
            for q in range(grp):
                @pl.when(live(g0, q))
                def _(q=q):
                    gather_desc(g0, q, 0).wait()
                    store_desc(g0, q, 0).start()
            for q in range(grp):
                @pl.when(live(g1, q))
                def _(q=q):
                    gather_desc(g1, q, 1).wait()
                    store_desc(g1, q, 1).start()
            return carry

        lax.fori_loop(0, npairs, body, 0)
        gl0 = (npairs - 1) * 2
        for q in range(grp):
            @pl.when(live(gl0, q))
            def _(q=q):
                store_desc(gl0, q, 0).wait()
        for q in range(grp):
            @pl.when(live(gl0 + 1, q))
            def _(q=q):
                store_desc(gl0 + 1, q, 1).wait()

    return gk(table, idxp)


def _sc_scatter(msg, idxp, R, n_nodes, width, with_count, grp=GRP):
    rows_per = idxp.shape[0] // NW
    npairs = rows_per // (2 * grp)
    npsub = n_nodes // 16
    mesh = plsc.VectorSubcoreMesh(core_axis_name="c", subcore_axis_name="s")

    zacc = jnp.zeros((n_nodes, width), jnp.float32)
    out_type = [jax.ShapeDtypeStruct((2, n_nodes, width), jnp.float32)]
    scratch = [
        pltpu.VMEM((rows_per, CHUNK), jnp.int32),
        pltpu.VMEM((2 * grp, CHUNK, width), jnp.float32),
        pltpu.VMEM_SHARED((n_nodes, width), jnp.float32),
    ]
    if with_count:
        zcnt = jnp.zeros((n_nodes, 8), jnp.float32)
        ones = jnp.ones((CHUNK, 8), jnp.float32)
        out_type.append(jax.ShapeDtypeStruct((2, n_nodes, 8), jnp.float32))
        scratch += [
            pltpu.VMEM((CHUNK, 8), jnp.float32),
            pltpu.VMEM_SHARED((n_nodes, 8), jnp.float32),
        ]
    scratch += [pltpu.SemaphoreType.DMA] * 6

    @functools.partial(
        pl.kernel, out_type=tuple(out_type), mesh=mesh, scratch_types=scratch,
        compiler_params=pltpu.CompilerParams(use_tc_tiling_on_sc=False),
    )
    def sk(*refs):
        if with_count:
            (m_hbm, idx_hbm, za_hbm, zc_hbm, ones_hbm, accp_hbm, cntp_hbm,
             idx_all, bufs, acc_sh, ones_v, cnt_sh,
             sl0, sl1, sa0, sa1, sc0, sc1) = refs
        else:
            (m_hbm, idx_hbm, za_hbm, accp_hbm,
             idx_all, bufs, acc_sh, sl0, sl1, sa0, sa1, sc0, sc1) = refs
        cid = lax.axis_index("c")
        sid = lax.axis_index("s")
        wid = sid * 2 + cid
        base = wid * rows_per

        pltpu.sync_copy(
            za_hbm.at[pl.ds(sid * npsub, npsub)],
            acc_sh.at[pl.ds(sid * npsub, npsub)],
        )
        if with_count:
            pltpu.sync_copy(
                zc_hbm.at[pl.ds(sid * npsub, npsub)],
                cnt_sh.at[pl.ds(sid * npsub, npsub)],
            )
            pltpu.sync_copy(ones_hbm, ones_v)
        pltpu.sync_copy(idx_hbm.at[pl.ds(base, rows_per)], idx_all)
        plsc.subcore_barrier()

        sls = (sl0, sl1)
        sas = (sa0, sa1)
        scs = (sc0, sc1)

        def load_desc(g, q, s):
            j = g * grp + q
            return pltpu.make_async_copy(
                m_hbm.at[pl.ds((base + j) * CHUNK, CHUNK)],
                bufs.at[s * grp + q],
                sls[s],
            )

        def add_desc(g, q, s):
            j = g * grp + q
            return pltpu.make_async_copy(
                bufs.at[s * grp + q], acc_sh.at[idx_all.at[j]], sas[s]
            )

        def cnt_desc(g, q, s):
            j = g * grp + q
            return pltpu.make_async_copy(
                ones_v, cnt_sh.at[idx_all.at[j]], scs[s]
            )

        def live(g, q):
            return base + g * grp + q < R

        def body(jp, carry):
            g0 = jp * 2
            g1 = g0 + 1
            for q in range(grp):
                @pl.when(jnp.logical_and(g0 >= 2, live(g0 - 2, q)))
                def _(q=q):
                    add_desc(g0 - 2, q, 0).wait()
                    if with_count:
                        cnt_desc(g0 - 2, q, 0).wait()
            for q in range(grp):
                @pl.when(live(g0, q))
                def _(q=q):
                    load_desc(g0, q, 0).start()
            for q in range(grp):
                @pl.when(jnp.logical_and(g0 >= 2, live(g0 - 1, q)))
                def _(q=q):
                    add_desc(g0 - 1, q, 1).wait()
                    if with_count:
                        cnt_desc(g0 - 1, q, 1).wait()
            for q in range(grp):
                @pl.when(live(g1, q))
                def _(q=q):
                    load_desc(g1, q, 1).start()
            for q in range(grp):
                @pl.when(live(g0, q))
                def _(q=q):
                    load_desc(g0, q, 0).wait()
                    add_desc(g0, q, 0).start(add=True)
                    if with_count:
                        cnt_desc(g0, q, 0).start(add=True)
            for q in range(grp):
                @pl.when(live(g1, q))
                def _(q=q):
                    load_desc(g1, q, 1).wait()
                    add_desc(g1, q, 1).start(add=True)
                    if with_count:
                        cnt_desc(g1, q, 1).start(add=True)
            return carry

        lax.fori_loop(0, npairs, body, 0)
        gl0 = (npairs - 1) * 2
        for q in range(grp):
            @pl.when(live(gl0, q))
            def _(q=q):
                add_desc(gl0, q, 0).wait()
                if with_count:
                    cnt_desc(gl0, q, 0).wait()
        for q in range(grp):
            @pl.when(live(gl0 + 1, q))
            def _(q=q):
                add_desc(gl0 + 1, q, 1).wait()
                if with_count:
                    cnt_desc(gl0 + 1, q, 1).wait()
        plsc.subcore_barrier()

        pltpu.sync_copy(
            acc_sh.at[pl.ds(sid * npsub, npsub)],
            accp_hbm.at[cid, pl.ds(sid * npsub, npsub)],
        )
        if with_count:
            pltpu.sync_copy(
                cnt_sh.at[pl.ds(sid * npsub, npsub)],
                cntp_hbm.at[cid, pl.ds(sid * npsub, npsub)],
            )

    if with_count:
        return sk(msg, idxp, zacc, zcnt, ones)
    return sk(msg, idxp, zacc)[0]


def _build_p1(fc1_W2, fc1_b2):
    w0 = fc1_W2[:, :256].reshape(16, 16, 16)
    w1 = fc1_W2[:, 256:384].reshape(16, 16, 8)
    w2 = fc1_W2[:, 384:512].reshape(16, 16, 8)
    p = jnp.concatenate([w0, w1, w2], axis=2)
    p = p.transpose(1, 0, 2).reshape(16, 512)
    b = jnp.concatenate(
        [
            fc1_b2[:256].reshape(16, 16),
            fc1_b2[256:384].reshape(16, 8),
            fc1_b2[384:512].reshape(16, 8),
        ],
        axis=1,
    )
    return jnp.concatenate([p, b], axis=1)


def _build_p2(fc2_W2, fc2_b2):
    w00 = fc2_W2[:, :256].reshape(16, 16, 16)
    w11 = fc2_W2[:, 256:384].reshape(16, 8, 16)
    w22 = fc2_W2[:, 384:512].reshape(16, 8, 16)
    p = jnp.concatenate([w00, w11, w22], axis=1)
    p = p.transpose(1, 0, 2).reshape(32, 256)
    b = jnp.concatenate(
        [
            fc2_b2[:256].reshape(16, 16),
            fc2_b2[256:384].reshape(8, 16),
            fc2_b2[384:512].reshape(8, 16),
        ],
        axis=0,
    )
    return jnp.concatenate([p, b], axis=1)


def kernel(node_feature, edge_index, edge_feature, edge_vec, W_node, b_node,
           W_skip, b_skip, fc1_W1, fc1_b1, fc1_W2, fc1_b2, fc2_W1, fc2_b1,
           fc2_W2, fc2_b2, bn_gamma, bn_beta, W_nl2, b_nl2):
    N, D = node_feature.shape
    E = edge_index.shape[1]
    R = E // CHUNK

    rows_per = -(-R // NW)
    rows_per += (-rows_per) % (2 * GRP)
    pad = NW * rows_per - R
    src2d = jnp.pad(edge_index[0].reshape(R, CHUNK).astype(jnp.int32),
                    ((0, pad), (0, 0)))
    dst2d = jnp.pad(edge_index[1].reshape(R, CHUNK).astype(jnp.int32),
                    ((0, pad), (0, 0)))

    p1t = _build_p1(fc1_W2, fc1_b2).T
    p2t = _build_p2(fc2_W2, fc2_b2).T
    w1at = fc1_W1.T
    w2at = fc2_W1.T
    bn2 = b_node.reshape(1, NSC)
    bs2 = b_skip.reshape(1, D)
    b1c = fc1_b1.reshape(16, 1)
    b2c = fc2_b1.reshape(16, 1)
    gamma = bn_gamma.reshape(1, NSC)
    beta = bn_beta.reshape(1, NSC)
    bnl = b_nl2.reshape(1, D)

    BN_ = 1000
    h, skip = pl.pallas_call(
        _node_body,
        grid=(N // BN_,),
        in_specs=[
            pl.BlockSpec((BN_, D), lambda i: (i, 0)),
            pl.BlockSpec((D, NSC), lambda i: (0, 0)),
            pl.BlockSpec((1, NSC), lambda i: (0, 0)),
            pl.BlockSpec((D, D), lambda i: (0, 0)),
            pl.BlockSpec((1, D), lambda i: (0, 0)),
        ],
        out_specs=[
            pl.BlockSpec((BN_, NSC), lambda i: (i, 0)),
            pl.BlockSpec((BN_, D), lambda i: (i, 0)),
        ],
        out_shape=[
            jax.ShapeDtypeStruct((N, NSC), jnp.float32),
            jax.ShapeDtypeStruct((N, D), jnp.float32),
        ],
    )(node_feature, W_node, bn2, W_skip, bs2)

    xh = _sc_gather(h, dst2d, R, NSC)

    BE = 3200
    m1 = pl.pallas_call(
        _edge1_body,
        grid=(E // BE,),
        in_specs=[
            pl.BlockSpec((BE, 16), lambda i: (i, 0)),
            pl.BlockSpec((BE, 3), lambda i: (i, 0)),
            pl.BlockSpec((BE, NSC), lambda i: (i, 0)),
            pl.BlockSpec((16, 16), lambda i: (0, 0)),
            pl.BlockSpec((16, 1), lambda i: (0, 0)),
            pl.BlockSpec((544, 16), lambda i: (0, 0)),
        ],
        out_specs=pl.BlockSpec((BE, F1), lambda i: (i, 0)),
        out_shape=jax.ShapeDtypeStruct((E, F1), jnp.float32),
    )(edge_feature, edge_vec, xh, w1at, b1c, p1t)

    o1p, cntp = _sc_scatter(m1, src2d, R, N, F1, with_count=True, grp=2)

    BF = 1000
    o1 = pl.pallas_call(
        _fin1_body,
        grid=(N // BF,),
        in_specs=[
            pl.BlockSpec((2, BF, F1), lambda i: (0, i, 0)),
            pl.BlockSpec((2, BF, 8), lambda i: (0, i, 0)),
            pl.BlockSpec((BF, NSC), lambda i: (i, 0)),
        ],
        out_specs=pl.BlockSpec((BF, F1), lambda i: (i, 0)),
        out_shape=jax.ShapeDtypeStruct((N, F1), jnp.float32),
    )(o1p, cntp, h)

    xo = _sc_gather(o1, dst2d, R, F1)

    m2 = pl.pallas_call(
        _edge2_body,
        grid=(E // BE,),
        in_specs=[
            pl.BlockSpec((BE, 16), lambda i: (i, 0)),
            pl.BlockSpec((BE, 3), lambda i: (i, 0)),
            pl.BlockSpec((BE, F1), lambda i: (i, 0)),
            pl.BlockSpec((16, 16), lambda i: (0, 0)),
            pl.BlockSpec((16, 1), lambda i: (0, 0)),
            pl.BlockSpec((272, 32), lambda i: (0, 0)),
        ],
        out_specs=pl.BlockSpec((BE, NSC), lambda i: (i, 0)),
        out_shape=jax.ShapeDtypeStruct((E, NSC), jnp.float32),
    )(edge_feature, edge_vec, xo, w2at, b2c, p2t)

    o2p = _sc_scatter(m2, src2d, R, N, NSC, with_count=False)

    BP = 1000
    out = pl.pallas_call(
        functools.partial(_epi_body, N),
        grid=(2, N // BP),
        in_specs=[
            pl.BlockSpec((2, BP, NSC), lambda p, i: (0, i, 0)),
            pl.BlockSpec((2, BP, 8), lambda p, i: (0, i, 0)),
            pl.BlockSpec((BP, D), lambda p, i: (i, 0)),
            pl.BlockSpec((1, NSC), lambda p, i: (0, 0)),
            pl.BlockSpec((1, NSC), lambda p, i: (0, 0)),
            pl.BlockSpec((NSC, D), lambda p, i: (0, 0)),
            pl.BlockSpec((1, D), lambda p, i: (0, 0)),
        ],
        out_specs=pl.BlockSpec((BP, D), lambda p, i: (i, 0)),
        out_shape=jax.ShapeDtypeStruct((N, D), jnp.float32),
        scratch_shapes=[pltpu.VMEM((2, NSC), jnp.float32)],
    )(o2p, cntp, skip, gamma, beta, W_nl2, bnl)

    return out


# speedup vs baseline: 3.9299x; 1.0256x over previous
import functools

import jax
import jax.numpy as jnp
from jax import lax
from jax.experimental import pallas as pl
from jax.experimental.pallas import tpu as pltpu
from jax.experimental.pallas import tpu_sc as plsc

NSC = 16
NVC = 8
F1 = NSC + 3 * NVC + 5 * NVC
C1 = float(NSC) ** -0.5
C2 = float(NSC + 2 * NVC) ** -0.5
S3 = 3.0 ** 0.5
S5 = 5.0 ** 0.5

NW = 32
CHUNK = 128


def _softplus(x):
    return jnp.maximum(x, 0.0) + jnp.log(1.0 + jnp.exp(-jnp.abs(x)))


def _sh_cols(ev):
    x = ev[0:1, :]
    y = ev[1:2, :]
    z = ev[2:3, :]
    inv = 1.0 / (jnp.sqrt(x * x + y * y + z * z) + 1e-12)
    xn = x * inv
    yn = y * inv
    zn = z * inv
    sh1 = [S3 * xn, S3 * yn, S3 * zn]
    sh2 = [
        S5 * S3 * xn * zn,
        S5 * S3 * xn * yn,
        S5 * (yn * yn - 0.5 * (xn * xn + zn * zn)),
        S5 * S3 * yn * zn,
        S5 * (S3 / 2.0) * (zn * zn - xn * xn),
    ]
    return sh1, sh2


def _node_body(nf, wn, bn, ws, bs, h_ref, skip_ref):
    x = nf[...]
    h_ref[...] = jnp.dot(x, wn[...], preferred_element_type=jnp.float32) + bn[...]
    skip_ref[...] = jnp.dot(x, ws[...], preferred_element_type=jnp.float32) + bs[...]


def _edge1_body(ef, ev, xh, w1at, b1c, p1t, m1_ref):
    sh1, sh2 = _sh_cols(ev[...].T)
    t = _softplus(
        jnp.dot(w1at[...], ef[...].T, preferred_element_type=jnp.float32)
        + b1c[...]
    )
    xt = xh[...].T
    a = jnp.dot(p1t[...], xt, preferred_element_type=jnp.float32)
    acc = a[512:544, :]
    for k in range(16):
        acc = acc + t[k : k + 1, :] * a[k * 32 : (k + 1) * 32, :]
    pre = acc * C1
    pre0 = pre[0:16, :]
    pre1 = pre[16:24, :]
    pre2 = pre[24:32, :]
    blocks = [pre0]
    for m in range(3):
        blocks.append(pre1 * sh1[m])
    for m in range(5):
        blocks.append(pre2 * sh2[m])
    m1_ref[...] = jnp.concatenate(blocks, axis=0).T


def _fin1_body(o1p, cntp, h, o1_ref):
    s = o1p[0] + o1p[1]
    c = cntp[0][:, 0:1] + cntp[1][:, 0:1]
    o1 = s / jnp.maximum(c, 1.0)
    o1_ref[...] = jnp.concatenate([o1[:, 0:16] + h[...], o1[:, 16:80]], axis=1)


def _edge2_body(ef, ev, xo, w2at, b2c, p2t, m2_ref):
    sh1, sh2 = _sh_cols(ev[...].T)
    t = _softplus(
        jnp.dot(w2at[...], ef[...].T, preferred_element_type=jnp.float32)
        + b2c[...]
    )
    f = xo[...].T
    r0 = f[0:16, :]
    r1 = (
        f[16:24, :] * sh1[0] + f[24:32, :] * sh1[1] + f[32:40, :] * sh1[2]
    ) * (3.0 ** -0.5)
    r2 = (
        f[40:48, :] * sh2[0]
        + f[48:56, :] * sh2[1]
        + f[56:64, :] * sh2[2]
        + f[64:72, :] * sh2[3]
        + f[72:80, :] * sh2[4]
    ) * (5.0 ** -0.5)
    r = jnp.concatenate([r0, r1, r2], axis=0)
    a = jnp.dot(p2t[...], r, preferred_element_type=jnp.float32)
    acc = a[256:272, :]
    for k in range(16):
        acc = acc + t[k : k + 1, :] * a[k * 16 : (k + 1) * 16, :]
    m2_ref[...] = (acc * C2).T


def _epi_body(n_nodes, o2p, cntp, skip, gamma, beta, wnl, bnl, out_ref, stat_ref):
    ph = pl.program_id(0)
    i = pl.program_id(1)
    s = o2p[0] + o2p[1]
    c = cntp[0][:, 0:1] + cntp[1][:, 0:1]
    o2 = s / jnp.maximum(c, 1.0)

    @pl.when(jnp.logical_and(ph == 0, i == 0))
    def _():
        stat_ref[...] = jnp.zeros_like(stat_ref)

    @pl.when(ph == 0)
    def _():
        stat_ref[...] = stat_ref[...] + jnp.concatenate(
            [
                jnp.sum(o2, axis=0, keepdims=True),
                jnp.sum(o2 * o2, axis=0, keepdims=True),
            ],
            axis=0,
        )

    @pl.when(ph == 1)
    def _():
        inv_n = 1.0 / float(n_nodes)
        mu = stat_ref[0:1, :] * inv_n
        var = stat_ref[1:2, :] * inv_n - mu * mu
        xb = (o2 - mu) / jnp.sqrt(var + 1e-5) * gamma[...] + beta[...]
        y = _softplus(
            jnp.dot(_softplus(xb), wnl[...], preferred_element_type=jnp.float32)
            + bnl[...]
        )
        out_ref[...] = y + skip[...]


GRP = 4


def _sc_gather(table, idxp, R, width, grp=GRP):
    rows_per = idxp.shape[0] // NW
    npairs = rows_per // (2 * grp)
    mesh = plsc.VectorSubcoreMesh(core_axis_name="c", subcore_axis_name="s")

    @functools.partial(
        pl.kernel,
        out_type=jax.ShapeDtypeStruct((R * CHUNK, width), jnp.float32),
        mesh=mesh,
        scratch_types=[
            pltpu.VMEM((rows_per, CHUNK), jnp.int32),
            pltpu.VMEM((2 * grp, CHUNK, width), jnp.float32),
            pltpu.SemaphoreType.DMA,
            pltpu.SemaphoreType.DMA,
            pltpu.SemaphoreType.DMA,
            pltpu.SemaphoreType.DMA,
        ],
        compiler_params=pltpu.CompilerParams(use_tc_tiling_on_sc=False),
    )
    def gk(table_hbm, idx_hbm, out_hbm, idx_all, bufs, sg0, sg1, ss0, ss1):
        cid = lax.axis_index("c")
        sid = lax.axis_index("s")
        wid = sid * 2 + cid
        base = wid * rows_per
        pltpu.sync_copy(idx_hbm.at[pl.ds(base, rows_per)], idx_all)
        sgs = (sg0, sg1)
        sss = (ss0, ss1)

        def gather_desc(g, q, s):
            j = g * grp + q
            return pltpu.make_async_copy(
                table_hbm.at[idx_all.at[j]], bufs.at[s * grp + q], sgs[s]
            )

        def store_desc(g, q, s):
            j = g * grp + q
            return pltpu.make_async_copy(
                bufs.at[s * grp + q],
                out_hbm.at[pl.ds((base + j) * CHUNK, CHUNK)],
                sss[s],
            )

        def live(g, q):
            return base + g * grp + q < R

        def body(jp, carry):
            g0 = jp * 2
            g1 = g0 + 1
            for q in range(grp):
                @pl.when(jnp.logical_and(g0 >= 2, live(g0 - 2, q)))
                def _(q=q):
                    store_desc(g0 - 2, q, 0).wait()
            for q in range(grp):
                @pl.when(live(g0, q))
                def _(q=q):
                    gather_desc(g0, q, 0).start()
            for q in range(grp):
                @pl.when(jnp.logical_and(g0 >= 2, live(g0 - 1, q)))
                def _(q=q):
                    store_desc(g0 - 1, q, 1).wait()
            for q in range(grp):
                @pl.when(live(g1, q))
                def _(q=q):
                    gather_desc(g1, q, 1).start()
            for q in range(grp):
                @pl.when(live(g0, q))
                def _(q=q):
                    gather_desc(g0, q, 0).wait()
                    store_desc(g0, q, 0).start()
            for q in range(grp):
                @pl.when(live(g1, q))
                def _(q=q):
                    gather_desc(g1, q, 1).wait()
                    store_desc(g1, q, 1).start()
            return carry

        lax.fori_loop(0, npairs, body, 0)
        gl0 = (npairs - 1) * 2
        for q in range(grp):
            @pl.when(live(gl0, q))
            def _(q=q):
                store_desc(gl0, q, 0).wait()
        for q in range(grp):
            @pl.when(live(gl0 + 1, q))
            def _(q=q):
                store_desc(gl0 + 1, q, 1).wait()

    return gk(table, idxp)


def _sc_scatter(msg, idxp, R, n_nodes, width, with_count, grp=GRP):
    rows_per = idxp.shape[0] // NW
    npairs = rows_per // (2 * grp)
    npsub = n_nodes // 16
    mesh = plsc.VectorSubcoreMesh(core_axis_name="c", subcore_axis_name="s")

    zacc = jnp.zeros((n_nodes, width), jnp.float32)
    out_type = [jax.ShapeDtypeStruct((2, n_nodes, width), jnp.float32)]
    scratch = [
        pltpu.VMEM((rows_per, CHUNK), jnp.int32),
        pltpu.VMEM((2 * grp, CHUNK, width), jnp.float32),
        pltpu.VMEM_SHARED((n_nodes, width), jnp.float32),
    ]
    if with_count:
        zcnt = jnp.zeros((n_nodes, 8), jnp.float32)
        ones = jnp.ones((CHUNK, 8), jnp.float32)
        out_type.append(jax.ShapeDtypeStruct((2, n_nodes, 8), jnp.float32))
        scratch += [
            pltpu.VMEM((CHUNK, 8), jnp.float32),
            pltpu.VMEM_SHARED((n_nodes, 8), jnp.float32),
        ]
    scratch += [pltpu.SemaphoreType.DMA] * 6

    @functools.partial(
        pl.kernel, out_type=tuple(out_type), mesh=mesh, scratch_types=scratch,
        compiler_params=pltpu.CompilerParams(use_tc_tiling_on_sc=False),
    )
    def sk(*refs):
        if with_count:
            (m_hbm, idx_hbm, za_hbm, zc_hbm, ones_hbm, accp_hbm, cntp_hbm,
             idx_all, bufs, acc_sh, ones_v, cnt_sh,
             sl0, sl1, sa0, sa1, sc0, sc1) = refs
        else:
            (m_hbm, idx_hbm, za_hbm, accp_hbm,
             idx_all, bufs, acc_sh, sl0, sl1, sa0, sa1, sc0, sc1) = refs
        cid = lax.axis_index("c")
        sid = lax.axis_index("s")
        wid = sid * 2 + cid
        base = wid * rows_per

        pltpu.sync_copy(
            za_hbm.at[pl.ds(sid * npsub, npsub)],
            acc_sh.at[pl.ds(sid * npsub, npsub)],
        )
        if with_count:
            pltpu.sync_copy(
                zc_hbm.at[pl.ds(sid * npsub, npsub)],
                cnt_sh.at[pl.ds(sid * npsub, npsub)],
            )
            pltpu.sync_copy(ones_hbm, ones_v)
        pltpu.sync_copy(idx_hbm.at[pl.ds(base, rows_per)], idx_all)
        plsc.subcore_barrier()

        sls = (sl0, sl1)
        sas = (sa0, sa1)
        scs = (sc0, sc1)

        def load_desc(g, q, s):
            j = g * grp + q
            return pltpu.make_async_copy(
                m_hbm.at[pl.ds((base + j) * CHUNK, CHUNK)],
                bufs.at[s * grp + q],
                sls[s],
            )

        def add_desc(g, q, s):
            j = g * grp + q
            return pltpu.make_async_copy(
                bufs.at[s * grp + q], acc_sh.at[idx_all.at[j]], sas[s]
            )

        def cnt_desc(g, q, s):
            j = g * grp + q
            return pltpu.make_async_copy(
                ones_v, cnt_sh.at[idx_all.at[j]], scs[s]
            )

        def live(g, q):
            return base + g * grp + q < R

        def body(jp, carry):
            g0 = jp * 2
            g1 = g0 + 1
            for q in range(grp):
                @pl.when(jnp.logical_and(g0 >= 2, live(g0 - 2, q)))
                def _(q=q):
                    add_desc(g0 - 2, q, 0).wait()
                    if with_count:
                        cnt_desc(g0 - 2, q, 0).wait()
            for q in range(grp):
                @pl.when(live(g0, q))
                def _(q=q):
                    load_desc(g0, q, 0).start()
            for q in range(grp):
                @pl.when(jnp.logical_and(g0 >= 2, live(g0 - 1, q)))
                def _(q=q):
                    add_desc(g0 - 1, q, 1).wait()
                    if with_count:
                        cnt_desc(g0 - 1, q, 1).wait()
            for q in range(grp):
                @pl.when(live(g1, q))
                def _(q=q):
                    load_desc(g1, q, 1).start()
            for q in range(grp):
                @pl.when(live(g0, q))
                def _(q=q):
                    load_desc(g0, q, 0).wait()
                    add_desc(g0, q, 0).start(add=True)
                    if with_count:
                        cnt_desc(g0, q, 0).start(add=True)
            for q in range(grp):
                @pl.when(live(g1, q))
                def _(q=q):
                    load_desc(g1, q, 1).wait()
                    add_desc(g1, q, 1).start(add=True)
                    if with_count:
                        cnt_desc(g1, q, 1).start(add=True)
            return carry

        lax.fori_loop(0, npairs, body, 0)
        gl0 = (npairs - 1) * 2
        for q in range(grp):
            @pl.when(live(gl0, q))
            def _(q=q):
                add_desc(gl0, q, 0).wait()
                if with_count:
                    cnt_desc(gl0, q, 0).wait()
        for q in range(grp):
            @pl.when(live(gl0 + 1, q))
            def _(q=q):
                add_desc(gl0 + 1, q, 1).wait()
                if with_count:
                    cnt_desc(gl0 + 1, q, 1).wait()
        plsc.subcore_barrier()

        pltpu.sync_copy(
            acc_sh.at[pl.ds(sid * npsub, npsub)],
            accp_hbm.at[cid, pl.ds(sid * npsub, npsub)],
        )
        if with_count:
            pltpu.sync_copy(
                cnt_sh.at[pl.ds(sid * npsub, npsub)],
                cntp_hbm.at[cid, pl.ds(sid * npsub, npsub)],
            )

    if with_count:
        return sk(msg, idxp, zacc, zcnt, ones)
    return sk(msg, idxp, zacc)[0]


def _build_p1(fc1_W2, fc1_b2):
    w0 = fc1_W2[:, :256].reshape(16, 16, 16)
    w1 = fc1_W2[:, 256:384].reshape(16, 16, 8)
    w2 = fc1_W2[:, 384:512].reshape(16, 16, 8)
    p = jnp.concatenate([w0, w1, w2], axis=2)
    p = p.transpose(1, 0, 2).reshape(16, 512)
    b = jnp.concatenate(
        [
            fc1_b2[:256].reshape(16, 16),
            fc1_b2[256:384].reshape(16, 8),
            fc1_b2[384:512].reshape(16, 8),
        ],
        axis=1,
    )
    return jnp.concatenate([p, b], axis=1)


def _build_p2(fc2_W2, fc2_b2):
    w00 = fc2_W2[:, :256].reshape(16, 16, 16)
    w11 = fc2_W2[:, 256:384].reshape(16, 8, 16)
    w22 = fc2_W2[:, 384:512].reshape(16, 8, 16)
    p = jnp.concatenate([w00, w11, w22], axis=1)
    p = p.transpose(1, 0, 2).reshape(32, 256)
    b = jnp.concatenate(
        [
            fc2_b2[:256].reshape(16, 16),
            fc2_b2[256:384].reshape(8, 16),
            fc2_b2[384:512].reshape(8, 16),
        ],
        axis=0,
    )
    return jnp.concatenate([p, b], axis=1)


def kernel(node_feature, edge_index, edge_feature, edge_vec, W_node, b_node,
           W_skip, b_skip, fc1_W1, fc1_b1, fc1_W2, fc1_b2, fc2_W1, fc2_b1,
           fc2_W2, fc2_b2, bn_gamma, bn_beta, W_nl2, b_nl2):
    N, D = node_feature.shape
    E = edge_index.shape[1]
    R = E // CHUNK

    rows_per = -(-R // NW)
    rows_per += (-rows_per) % (2 * GRP)
    pad = NW * rows_per - R
    src2d = jnp.pad(edge_index[0].reshape(R, CHUNK).astype(jnp.int32),
                    ((0, pad), (0, 0)))
    dst2d = jnp.pad(edge_index[1].reshape(R, CHUNK).astype(jnp.int32),
                    ((0, pad), (0, 0)))

    p1t = _build_p1(fc1_W2, fc1_b2).T
    p2t = _build_p2(fc2_W2, fc2_b2).T
    w1at = fc1_W1.T
    w2at = fc2_W1.T
    bn2 = b_node.reshape(1, NSC)
    bs2 = b_skip.reshape(1, D)
    b1c = fc1_b1.reshape(16, 1)
    b2c = fc2_b1.reshape(16, 1)
    gamma = bn_gamma.reshape(1, NSC)
    beta = bn_beta.reshape(1, NSC)
    bnl = b_nl2.reshape(1, D)

    BN_ = 1000
    h, skip = pl.pallas_call(
        _node_body,
        grid=(N // BN_,),
        in_specs=[
            pl.BlockSpec((BN_, D), lambda i: (i, 0)),
            pl.BlockSpec((D, NSC), lambda i: (0, 0)),
            pl.BlockSpec((1, NSC), lambda i: (0, 0)),
            pl.BlockSpec((D, D), lambda i: (0, 0)),
            pl.BlockSpec((1, D), lambda i: (0, 0)),
        ],
        out_specs=[
            pl.BlockSpec((BN_, NSC), lambda i: (i, 0)),
            pl.BlockSpec((BN_, D), lambda i: (i, 0)),
        ],
        out_shape=[
            jax.ShapeDtypeStruct((N, NSC), jnp.float32),
            jax.ShapeDtypeStruct((N, D), jnp.float32),
        ],
    )(node_feature, W_node, bn2, W_skip, bs2)

    xh = _sc_gather(h, dst2d, R, NSC)

    BE = 8000
    m1 = pl.pallas_call(
        _edge1_body,
        grid=(E // BE,),
        in_specs=[
            pl.BlockSpec((BE, 16), lambda i: (i, 0)),
            pl.BlockSpec((BE, 3), lambda i: (i, 0)),
            pl.BlockSpec((BE, NSC), lambda i: (i, 0)),
            pl.BlockSpec((16, 16), lambda i: (0, 0)),
            pl.BlockSpec((16, 1), lambda i: (0, 0)),
            pl.BlockSpec((544, 16), lambda i: (0, 0)),
        ],
        out_specs=pl.BlockSpec((BE, F1), lambda i: (i, 0)),
        out_shape=jax.ShapeDtypeStruct((E, F1), jnp.float32),
    )(edge_feature, edge_vec, xh, w1at, b1c, p1t)

    o1p, cntp = _sc_scatter(m1, src2d, R, N, F1, with_count=True, grp=2)

    BF = 1000
    o1 = pl.pallas_call(
        _fin1_body,
        grid=(N // BF,),
        in_specs=[
            pl.BlockSpec((2, BF, F1), lambda i: (0, i, 0)),
            pl.BlockSpec((2, BF, 8), lambda i: (0, i, 0)),
            pl.BlockSpec((BF, NSC), lambda i: (i, 0)),
        ],
        out_specs=pl.BlockSpec((BF, F1), lambda i: (i, 0)),
        out_shape=jax.ShapeDtypeStruct((N, F1), jnp.float32),
    )(o1p, cntp, h)

    xo = _sc_gather(o1, dst2d, R, F1)

    m2 = pl.pallas_call(
        _edge2_body,
        grid=(E // BE,),
        in_specs=[
            pl.BlockSpec((BE, 16), lambda i: (i, 0)),
            pl.BlockSpec((BE, 3), lambda i: (i, 0)),
            pl.BlockSpec((BE, F1), lambda i: (i, 0)),
            pl.BlockSpec((16, 16), lambda i: (0, 0)),
            pl.BlockSpec((16, 1), lambda i: (0, 0)),
            pl.BlockSpec((272, 32), lambda i: (0, 0)),
        ],
        out_specs=pl.BlockSpec((BE, NSC), lambda i: (i, 0)),
        out_shape=jax.ShapeDtypeStruct((E, NSC), jnp.float32),
    )(edge_feature, edge_vec, xo, w2at, b2c, p2t)

    o2p = _sc_scatter(m2, src2d, R, N, NSC, with_count=False)

    BP = 1000
    out = pl.pallas_call(
        functools.partial(_epi_body, N),
        grid=(2, N // BP),
        in_specs=[
            pl.BlockSpec((2, BP, NSC), lambda p, i: (0, i, 0)),
            pl.BlockSpec((2, BP, 8), lambda p, i: (0, i, 0)),
            pl.BlockSpec((BP, D), lambda p, i: (i, 0)),
            pl.BlockSpec((1, NSC), lambda p, i: (0, 0)),
            pl.BlockSpec((1, NSC), lambda p, i: (0, 0)),
            pl.BlockSpec((NSC, D), lambda p, i: (0, 0)),
            pl.BlockSpec((1, D), lambda p, i: (0, 0)),
        ],
        out_specs=pl.BlockSpec((BP, D), lambda p, i: (i, 0)),
        out_shape=jax.ShapeDtypeStruct((N, D), jnp.float32),
        scratch_shapes=[pltpu.VMEM((2, NSC), jnp.float32)],
    )(o2p, cntp, skip, gamma, beta, W_nl2, bnl)

    return out


# speedup vs baseline: 3.9798x; 1.0127x over previous
import functools

import jax
import jax.numpy as jnp
from jax import lax
from jax.experimental import pallas as pl
from jax.experimental.pallas import tpu as pltpu
from jax.experimental.pallas import tpu_sc as plsc

NSC = 16
NVC = 8
F1 = NSC + 3 * NVC + 5 * NVC
C1 = float(NSC) ** -0.5
C2 = float(NSC + 2 * NVC) ** -0.5
S3 = 3.0 ** 0.5
S5 = 5.0 ** 0.5

NW = 32
CHUNK = 128


def _softplus(x):
    return jnp.maximum(x, 0.0) + jnp.log(1.0 + jnp.exp(-jnp.abs(x)))


def _sh_cols(ev):
    x = ev[0:1, :]
    y = ev[1:2, :]
    z = ev[2:3, :]
    inv = 1.0 / (jnp.sqrt(x * x + y * y + z * z) + 1e-12)
    xn = x * inv
    yn = y * inv
    zn = z * inv
    sh1 = [S3 * xn, S3 * yn, S3 * zn]
    sh2 = [
        S5 * S3 * xn * zn,
        S5 * S3 * xn * yn,
        S5 * (yn * yn - 0.5 * (xn * xn + zn * zn)),
        S5 * S3 * yn * zn,
        S5 * (S3 / 2.0) * (zn * zn - xn * xn),
    ]
    return sh1, sh2


def _node_body(nf, wn, bn, ws, bs, h_ref, skip_ref):
    x = nf[...]
    h_ref[...] = jnp.dot(x, wn[...], preferred_element_type=jnp.float32) + bn[...]
    skip_ref[...] = jnp.dot(x, ws[...], preferred_element_type=jnp.float32) + bs[...]


def _edge1_body(ef, ev, xh, w1at, b1c, p1t, m1_ref):
    sh1, sh2 = _sh_cols(ev[...].T)
    t = _softplus(
        jnp.dot(w1at[...], ef[...].T, preferred_element_type=jnp.float32)
        + b1c[...]
    )
    xt = xh[...].T
    a = jnp.dot(p1t[...], xt, preferred_element_type=jnp.float32)
    acc = a[512:544, :]
    for k in range(16):
        acc = acc + t[k : k + 1, :] * a[k * 32 : (k + 1) * 32, :]
    pre = acc * C1
    pre0 = pre[0:16, :]
    pre1 = pre[16:24, :]
    pre2 = pre[24:32, :]
    blocks = [pre0]
    for m in range(3):
        blocks.append(pre1 * sh1[m])
    for m in range(5):
        blocks.append(pre2 * sh2[m])
    m1_ref[...] = jnp.concatenate(blocks, axis=0).T


def _fin1_body(o1p, cntp, h, o1_ref):
    s = o1p[0] + o1p[1]
    c = cntp[0][:, 0:1] + cntp[1][:, 0:1]
    o1 = s / jnp.maximum(c, 1.0)
    o1_ref[...] = jnp.concatenate([o1[:, 0:16] + h[...], o1[:, 16:80]], axis=1)


def _edge2_body(ef, ev, xo, w2at, b2c, p2t, m2_ref):
    sh1, sh2 = _sh_cols(ev[...].T)
    t = _softplus(
        jnp.dot(w2at[...], ef[...].T, preferred_element_type=jnp.float32)
        + b2c[...]
    )
    f = xo[...].T
    r0 = f[0:16, :]
    r1 = (
        f[16:24, :] * sh1[0] + f[24:32, :] * sh1[1] + f[32:40, :] * sh1[2]
    ) * (3.0 ** -0.5)
    r2 = (
        f[40:48, :] * sh2[0]
        + f[48:56, :] * sh2[1]
        + f[56:64, :] * sh2[2]
        + f[64:72, :] * sh2[3]
        + f[72:80, :] * sh2[4]
    ) * (5.0 ** -0.5)
    r = jnp.concatenate([r0, r1, r2], axis=0)
    a = jnp.dot(p2t[...], r, preferred_element_type=jnp.float32)
    acc = a[256:272, :]
    for k in range(16):
        acc = acc + t[k : k + 1, :] * a[k * 16 : (k + 1) * 16, :]
    m2_ref[...] = (acc * C2).T


def _epi_body(n_nodes, o2p, cntp, skip, gamma, beta, wnl, bnl, out_ref, stat_ref):
    ph = pl.program_id(0)
    i = pl.program_id(1)
    s = o2p[0] + o2p[1]
    c = cntp[0][:, 0:1] + cntp[1][:, 0:1]
    o2 = s / jnp.maximum(c, 1.0)

    @pl.when(jnp.logical_and(ph == 0, i == 0))
    def _():
        stat_ref[...] = jnp.zeros_like(stat_ref)

    @pl.when(ph == 0)
    def _():
        stat_ref[...] = stat_ref[...] + jnp.concatenate(
            [
                jnp.sum(o2, axis=0, keepdims=True),
                jnp.sum(o2 * o2, axis=0, keepdims=True),
            ],
            axis=0,
        )

    @pl.when(ph == 1)
    def _():
        inv_n = 1.0 / float(n_nodes)
        mu = stat_ref[0:1, :] * inv_n
        var = stat_ref[1:2, :] * inv_n - mu * mu
        xb = (o2 - mu) / jnp.sqrt(var + 1e-5) * gamma[...] + beta[...]
        y = _softplus(
            jnp.dot(_softplus(xb), wnl[...], preferred_element_type=jnp.float32)
            + bnl[...]
        )
        out_ref[...] = y + skip[...]


GRP = 4


def _sc_gather(table, idxp, R, width, grp=GRP):
    rows_per = idxp.shape[0] // NW
    npairs = rows_per // (2 * grp)
    mesh = plsc.VectorSubcoreMesh(core_axis_name="c", subcore_axis_name="s")

    @functools.partial(
        pl.kernel,
        out_type=jax.ShapeDtypeStruct((R * CHUNK, width), jnp.float32),
        mesh=mesh,
        scratch_types=[
            pltpu.VMEM((rows_per, CHUNK), jnp.int32),
            pltpu.VMEM((2 * grp, CHUNK, width), jnp.float32),
            pltpu.SemaphoreType.DMA,
            pltpu.SemaphoreType.DMA,
            pltpu.SemaphoreType.DMA,
            pltpu.SemaphoreType.DMA,
        ],
        compiler_params=pltpu.CompilerParams(use_tc_tiling_on_sc=False),
    )
    def gk(table_hbm, idx_hbm, out_hbm, idx_all, bufs, sg0, sg1, ss0, ss1):
        cid = lax.axis_index("c")
        sid = lax.axis_index("s")
        wid = sid * 2 + cid
        base = wid * rows_per
        pltpu.sync_copy(idx_hbm.at[pl.ds(base, rows_per)], idx_all)
        sgs = (sg0, sg1)
        sss = (ss0, ss1)

        def gather_desc(g, q, s):
            j = g * grp + q
            return pltpu.make_async_copy(
                table_hbm.at[idx_all.at[j]], bufs.at[s * grp + q], sgs[s]
            )

        def store_desc(g, q, s):
            j = g * grp + q
            return pltpu.make_async_copy(
                bufs.at[s * grp + q],
                out_hbm.at[pl.ds((base + j) * CHUNK, CHUNK)],
                sss[s],
            )

        def live(g, q):
            return base + g * grp + q < R

        def body(jp, carry):
            g0 = jp * 2
            g1 = g0 + 1
            for q in range(grp):
                @pl.when(jnp.logical_and(g0 >= 2, live(g0 - 2, q)))
                def _(q=q):
                    store_desc(g0 - 2, q, 0).wait()
            for q in range(grp):
                @pl.when(live(g0, q))
                def _(q=q):
                    gather_desc(g0, q, 0).start()
            for q in range(grp):
                @pl.when(jnp.logical_and(g0 >= 2, live(g0 - 1, q)))
                def _(q=q):
                    store_desc(g0 - 1, q, 1).wait()
            for q in range(grp):
                @pl.when(live(g1, q))
                def _(q=q):
                    gather_desc(g1, q, 1).start()
            for q in range(grp):
                @pl.when(live(g0, q))
                def _(q=q):
                    gather_desc(g0, q, 0).wait()
                    store_desc(g0, q, 0).start()
            for q in range(grp):
                @pl.when(live(g1, q))
                def _(q=q):
                    gather_desc(g1, q, 1).wait()
                    store_desc(g1, q, 1).start()
            return carry

        lax.fori_loop(0, npairs, body, 0)
        gl0 = (npairs - 1) * 2
        for q in range(grp):
            @pl.when(live(gl0, q))
            def _(q=q):
                store_desc(gl0, q, 0).wait()
        for q in range(grp):
            @pl.when(live(gl0 + 1, q))
            def _(q=q):
                store_desc(gl0 + 1, q, 1).wait()

    return gk(table, idxp)


def _sc_scatter(msg, idxp, R, n_nodes, width, with_count, grp=GRP):
    rows_per = idxp.shape[0] // NW
    npairs = rows_per // (2 * grp)
    npsub = n_nodes // 16
    mesh = plsc.VectorSubcoreMesh(core_axis_name="c", subcore_axis_name="s")

    zacc = jnp.zeros((n_nodes, width), jnp.float32)
    out_type = [jax.ShapeDtypeStruct((2, n_nodes, width), jnp.float32)]
    scratch = [
        pltpu.VMEM((rows_per, CHUNK), jnp.int32),
        pltpu.VMEM((2 * grp, CHUNK, width), jnp.float32),
        pltpu.VMEM_SHARED((n_nodes, width), jnp.float32),
    ]
    if with_count:
        zcnt = jnp.zeros((n_nodes, 8), jnp.float32)
        ones = jnp.ones((CHUNK, 8), jnp.float32)
        out_type.append(jax.ShapeDtypeStruct((2, n_nodes, 8), jnp.float32))
        scratch += [
            pltpu.VMEM((CHUNK, 8), jnp.float32),
            pltpu.VMEM_SHARED((n_nodes, 8), jnp.float32),
        ]
    scratch += [pltpu.SemaphoreType.DMA] * 6

    @functools.partial(
        pl.kernel, out_type=tuple(out_type), mesh=mesh, scratch_types=scratch,
        compiler_params=pltpu.CompilerParams(use_tc_tiling_on_sc=False),
    )
    def sk(*refs):
        if with_count:
            (m_hbm, idx_hbm, za_hbm, zc_hbm, ones_hbm, accp_hbm, cntp_hbm,
             idx_all, bufs, acc_sh, ones_v, cnt_sh,
             sl0, sl1, sa0, sa1, sc0, sc1) = refs
        else:
            (m_hbm, idx_hbm, za_hbm, accp_hbm,
             idx_all, bufs, acc_sh, sl0, sl1, sa0, sa1, sc0, sc1) = refs
        cid = lax.axis_index("c")
        sid = lax.axis_index("s")
        wid = sid * 2 + cid
        base = wid * rows_per

        pltpu.sync_copy(
            za_hbm.at[pl.ds(sid * npsub, npsub)],
            acc_sh.at[pl.ds(sid * npsub, npsub)],
        )
        if with_count:
            pltpu.sync_copy(
                zc_hbm.at[pl.ds(sid * npsub, npsub)],
                cnt_sh.at[pl.ds(sid * npsub, npsub)],
            )
            pltpu.sync_copy(ones_hbm, ones_v)
        pltpu.sync_copy(idx_hbm.at[pl.ds(base, rows_per)], idx_all)
        plsc.subcore_barrier()

        sls = (sl0, sl1)
        sas = (sa0, sa1)
        scs = (sc0, sc1)

        def load_desc(g, q, s):
            j = g * grp + q
            return pltpu.make_async_copy(
                m_hbm.at[pl.ds((base + j) * CHUNK, CHUNK)],
                bufs.at[s * grp + q],
                sls[s],
            )

        def add_desc(g, q, s):
            j = g * grp + q
            return pltpu.make_async_copy(
                bufs.at[s * grp + q], acc_sh.at[idx_all.at[j]], sas[s]
            )

        def cnt_desc(g, q, s):
            j = g * grp + q
            return pltpu.make_async_copy(
                ones_v, cnt_sh.at[idx_all.at[j]], scs[s]
            )

        def live(g, q):
            return base + g * grp + q < R

        def body(jp, carry):
            g0 = jp * 2
            g1 = g0 + 1
            for q in range(grp):
                @pl.when(jnp.logical_and(g0 >= 2, live(g0 - 2, q)))
                def _(q=q):
                    add_desc(g0 - 2, q, 0).wait()
                    if with_count:
                        cnt_desc(g0 - 2, q, 0).wait()
            for q in range(grp):
                @pl.when(live(g0, q))
                def _(q=q):
                    load_desc(g0, q, 0).start()
            for q in range(grp):
                @pl.when(jnp.logical_and(g0 >= 2, live(g0 - 1, q)))
                def _(q=q):
                    add_desc(g0 - 1, q, 1).wait()
                    if with_count:
                        cnt_desc(g0 - 1, q, 1).wait()
            for q in range(grp):
                @pl.when(live(g1, q))
                def _(q=q):
                    load_desc(g1, q, 1).start()
            for q in range(grp):
                @pl.when(live(g0, q))
                def _(q=q):
                    load_desc(g0, q, 0).wait()
                    add_desc(g0, q, 0).start(add=True)
                    if with_count:
                        cnt_desc(g0, q, 0).start(add=True)
            for q in range(grp):
                @pl.when(live(g1, q))
                def _(q=q):
                    load_desc(g1, q, 1).wait()
                    add_desc(g1, q, 1).start(add=True)
                    if with_count:
                        cnt_desc(g1, q, 1).start(add=True)
            return carry

        lax.fori_loop(0, npairs, body, 0)
        gl0 = (npairs - 1) * 2
        for q in range(grp):
            @pl.when(live(gl0, q))
            def _(q=q):
                add_desc(gl0, q, 0).wait()
                if with_count:
                    cnt_desc(gl0, q, 0).wait()
        for q in range(grp):
            @pl.when(live(gl0 + 1, q))
            def _(q=q):
                add_desc(gl0 + 1, q, 1).wait()
                if with_count:
                    cnt_desc(gl0 + 1, q, 1).wait()
        plsc.subcore_barrier()

        pltpu.sync_copy(
            acc_sh.at[pl.ds(sid * npsub, npsub)],
            accp_hbm.at[cid, pl.ds(sid * npsub, npsub)],
        )
        if with_count:
            pltpu.sync_copy(
                cnt_sh.at[pl.ds(sid * npsub, npsub)],
                cntp_hbm.at[cid, pl.ds(sid * npsub, npsub)],
            )

    if with_count:
        return sk(msg, idxp, zacc, zcnt, ones)
    return sk(msg, idxp, zacc)[0]


def _build_p1(fc1_W2, fc1_b2):
    w0 = fc1_W2[:, :256].reshape(16, 16, 16)
    w1 = fc1_W2[:, 256:384].reshape(16, 16, 8)
    w2 = fc1_W2[:, 384:512].reshape(16, 16, 8)
    p = jnp.concatenate([w0, w1, w2], axis=2)
    p = p.transpose(1, 0, 2).reshape(16, 512)
    b = jnp.concatenate(
        [
            fc1_b2[:256].reshape(16, 16),
            fc1_b2[256:384].reshape(16, 8),
            fc1_b2[384:512].reshape(16, 8),
        ],
        axis=1,
    )
    return jnp.concatenate([p, b], axis=1)


def _build_p2(fc2_W2, fc2_b2):
    w00 = fc2_W2[:, :256].reshape(16, 16, 16)
    w11 = fc2_W2[:, 256:384].reshape(16, 8, 16)
    w22 = fc2_W2[:, 384:512].reshape(16, 8, 16)
    p = jnp.concatenate([w00, w11, w22], axis=1)
    p = p.transpose(1, 0, 2).reshape(32, 256)
    b = jnp.concatenate(
        [
            fc2_b2[:256].reshape(16, 16),
            fc2_b2[256:384].reshape(8, 16),
            fc2_b2[384:512].reshape(8, 16),
        ],
        axis=0,
    )
    return jnp.concatenate([p, b], axis=1)


def kernel(node_feature, edge_index, edge_feature, edge_vec, W_node, b_node,
           W_skip, b_skip, fc1_W1, fc1_b1, fc1_W2, fc1_b2, fc2_W1, fc2_b1,
           fc2_W2, fc2_b2, bn_gamma, bn_beta, W_nl2, b_nl2):
    N, D = node_feature.shape
    E = edge_index.shape[1]
    R = E // CHUNK

    rows_per = -(-R // NW)
    rows_per += (-rows_per) % (2 * GRP)
    pad = NW * rows_per - R
    src2d = jnp.pad(edge_index[0].reshape(R, CHUNK).astype(jnp.int32),
                    ((0, pad), (0, 0)))
    dst2d = jnp.pad(edge_index[1].reshape(R, CHUNK).astype(jnp.int32),
                    ((0, pad), (0, 0)))

    p1t = _build_p1(fc1_W2, fc1_b2).T
    p2t = _build_p2(fc2_W2, fc2_b2).T
    w1at = fc1_W1.T
    w2at = fc2_W1.T
    bn2 = b_node.reshape(1, NSC)
    bs2 = b_skip.reshape(1, D)
    b1c = fc1_b1.reshape(16, 1)
    b2c = fc2_b1.reshape(16, 1)
    gamma = bn_gamma.reshape(1, NSC)
    beta = bn_beta.reshape(1, NSC)
    bnl = b_nl2.reshape(1, D)

    BN_ = 2000
    h, skip = pl.pallas_call(
        _node_body,
        grid=(N // BN_,),
        in_specs=[
            pl.BlockSpec((BN_, D), lambda i: (i, 0)),
            pl.BlockSpec((D, NSC), lambda i: (0, 0)),
            pl.BlockSpec((1, NSC), lambda i: (0, 0)),
            pl.BlockSpec((D, D), lambda i: (0, 0)),
            pl.BlockSpec((1, D), lambda i: (0, 0)),
        ],
        out_specs=[
            pl.BlockSpec((BN_, NSC), lambda i: (i, 0)),
            pl.BlockSpec((BN_, D), lambda i: (i, 0)),
        ],
        out_shape=[
            jax.ShapeDtypeStruct((N, NSC), jnp.float32),
            jax.ShapeDtypeStruct((N, D), jnp.float32),
        ],
    )(node_feature, W_node, bn2, W_skip, bs2)

    xh = _sc_gather(h, dst2d, R, NSC)

    BE = 8000
    m1 = pl.pallas_call(
        _edge1_body,
        grid=(E // BE,),
        in_specs=[
            pl.BlockSpec((BE, 16), lambda i: (i, 0)),
            pl.BlockSpec((BE, 3), lambda i: (i, 0)),
            pl.BlockSpec((BE, NSC), lambda i: (i, 0)),
            pl.BlockSpec((16, 16), lambda i: (0, 0)),
            pl.BlockSpec((16, 1), lambda i: (0, 0)),
            pl.BlockSpec((544, 16), lambda i: (0, 0)),
        ],
        out_specs=pl.BlockSpec((BE, F1), lambda i: (i, 0)),
        out_shape=jax.ShapeDtypeStruct((E, F1), jnp.float32),
    )(edge_feature, edge_vec, xh, w1at, b1c, p1t)

    o1p, cntp = _sc_scatter(m1, src2d, R, N, F1, with_count=True, grp=2)

    BF = 2000
    o1 = pl.pallas_call(
        _fin1_body,
        grid=(N // BF,),
        in_specs=[
            pl.BlockSpec((2, BF, F1), lambda i: (0, i, 0)),
            pl.BlockSpec((2, BF, 8), lambda i: (0, i, 0)),
            pl.BlockSpec((BF, NSC), lambda i: (i, 0)),
        ],
        out_specs=pl.BlockSpec((BF, F1), lambda i: (i, 0)),
        out_shape=jax.ShapeDtypeStruct((N, F1), jnp.float32),
    )(o1p, cntp, h)

    xo = _sc_gather(o1, dst2d, R, F1)

    m2 = pl.pallas_call(
        _edge2_body,
        grid=(E // BE,),
        in_specs=[
            pl.BlockSpec((BE, 16), lambda i: (i, 0)),
            pl.BlockSpec((BE, 3), lambda i: (i, 0)),
            pl.BlockSpec((BE, F1), lambda i: (i, 0)),
            pl.BlockSpec((16, 16), lambda i: (0, 0)),
            pl.BlockSpec((16, 1), lambda i: (0, 0)),
            pl.BlockSpec((272, 32), lambda i: (0, 0)),
        ],
        out_specs=pl.BlockSpec((BE, NSC), lambda i: (i, 0)),
        out_shape=jax.ShapeDtypeStruct((E, NSC), jnp.float32),
    )(edge_feature, edge_vec, xo, w2at, b2c, p2t)

    o2p = _sc_scatter(m2, src2d, R, N, NSC, with_count=False)

    BP = 2000
    out = pl.pallas_call(
        functools.partial(_epi_body, N),
        grid=(2, N // BP),
        in_specs=[
            pl.BlockSpec((2, BP, NSC), lambda p, i: (0, i, 0)),
            pl.BlockSpec((2, BP, 8), lambda p, i: (0, i, 0)),
            pl.BlockSpec((BP, D), lambda p, i: (i, 0)),
            pl.BlockSpec((1, NSC), lambda p, i: (0, 0)),
            pl.BlockSpec((1, NSC), lambda p, i: (0, 0)),
            pl.BlockSpec((NSC, D), lambda p, i: (0, 0)),
            pl.BlockSpec((1, D), lambda p, i: (0, 0)),
        ],
        out_specs=pl.BlockSpec((BP, D), lambda p, i: (i, 0)),
        out_shape=jax.ShapeDtypeStruct((N, D), jnp.float32),
        scratch_shapes=[pltpu.VMEM((2, NSC), jnp.float32)],
    )(o2p, cntp, skip, gamma, beta, W_nl2, bnl)

    return out


# speedup vs baseline: 3.9809x; 1.0003x over previous
import functools

import jax
import jax.numpy as jnp
from jax import lax
from jax.experimental import pallas as pl
from jax.experimental.pallas import tpu as pltpu
from jax.experimental.pallas import tpu_sc as plsc

NSC = 16
NVC = 8
F1 = NSC + 3 * NVC + 5 * NVC
C1 = float(NSC) ** -0.5
C2 = float(NSC + 2 * NVC) ** -0.5
S3 = 3.0 ** 0.5
S5 = 5.0 ** 0.5

NW = 32
CHUNK = 128


def _softplus(x):
    return jnp.maximum(x, 0.0) + jnp.log(1.0 + jnp.exp(-jnp.abs(x)))


def _sh_cols(ev):
    x = ev[0:1, :]
    y = ev[1:2, :]
    z = ev[2:3, :]
    inv = 1.0 / (jnp.sqrt(x * x + y * y + z * z) + 1e-12)
    xn = x * inv
    yn = y * inv
    zn = z * inv
    sh1 = [S3 * xn, S3 * yn, S3 * zn]
    sh2 = [
        S5 * S3 * xn * zn,
        S5 * S3 * xn * yn,
        S5 * (yn * yn - 0.5 * (xn * xn + zn * zn)),
        S5 * S3 * yn * zn,
        S5 * (S3 / 2.0) * (zn * zn - xn * xn),
    ]
    return sh1, sh2


def _node_body(nf, wn, bn, ws, bs, h_ref, skip_ref):
    x = nf[...]
    h_ref[...] = jnp.dot(x, wn[...], preferred_element_type=jnp.float32) + bn[...]
    skip_ref[...] = jnp.dot(x, ws[...], preferred_element_type=jnp.float32) + bs[...]


def _edge1_body(ef, ev, xh, w1at, b1c, p1t, m1_ref):
    sh1, sh2 = _sh_cols(ev[...].T)
    t = _softplus(
        jnp.dot(w1at[...], ef[...].T, preferred_element_type=jnp.float32)
        + b1c[...]
    )
    xt = xh[...].T
    a = jnp.dot(p1t[...], xt, preferred_element_type=jnp.float32)
    acc = a[512:544, :]
    for k in range(16):
        acc = acc + t[k : k + 1, :] * a[k * 32 : (k + 1) * 32, :]
    pre = acc * C1
    pre0 = pre[0:16, :]
    pre1 = pre[16:24, :]
    pre2 = pre[24:32, :]
    blocks = [pre0]
    for m in range(3):
        blocks.append(pre1 * sh1[m])
    for m in range(5):
        blocks.append(pre2 * sh2[m])
    m1_ref[...] = jnp.concatenate(blocks, axis=0).T


def _fin1_body(o1p, cntp, h, o1_ref):
    s = o1p[0] + o1p[1]
    c = cntp[0][:, 0:1] + cntp[1][:, 0:1]
    o1 = s / jnp.maximum(c, 1.0)
    o1_ref[...] = jnp.concatenate([o1[:, 0:16] + h[...], o1[:, 16:80]], axis=1)


def _edge2_body(ef, ev, xo, w2at, b2c, p2t, m2_ref):
    sh1, sh2 = _sh_cols(ev[...].T)
    t = _softplus(
        jnp.dot(w2at[...], ef[...].T, preferred_element_type=jnp.float32)
        + b2c[...]
    )
    f = xo[...].T
    r0 = f[0:16, :]
    r1 = (
        f[16:24, :] * sh1[0] + f[24:32, :] * sh1[1] + f[32:40, :] * sh1[2]
    ) * (3.0 ** -0.5)
    r2 = (
        f[40:48, :] * sh2[0]
        + f[48:56, :] * sh2[1]
        + f[56:64, :] * sh2[2]
        + f[64:72, :] * sh2[3]
        + f[72:80, :] * sh2[4]
    ) * (5.0 ** -0.5)
    r = jnp.concatenate([r0, r1, r2], axis=0)
    a = jnp.dot(p2t[...], r, preferred_element_type=jnp.float32)
    acc = a[256:272, :]
    for k in range(16):
        acc = acc + t[k : k + 1, :] * a[k * 16 : (k + 1) * 16, :]
    m2_ref[...] = (acc * C2).T


def _epi_body(n_nodes, o2p, cntp, skip, gamma, beta, wnl, bnl, out_ref, stat_ref):
    ph = pl.program_id(0)
    i = pl.program_id(1)
    s = o2p[0] + o2p[1]
    c = cntp[0][:, 0:1] + cntp[1][:, 0:1]
    o2 = s / jnp.maximum(c, 1.0)

    @pl.when(jnp.logical_and(ph == 0, i == 0))
    def _():
        stat_ref[...] = jnp.zeros_like(stat_ref)

    @pl.when(ph == 0)
    def _():
        stat_ref[...] = stat_ref[...] + jnp.concatenate(
            [
                jnp.sum(o2, axis=0, keepdims=True),
                jnp.sum(o2 * o2, axis=0, keepdims=True),
            ],
            axis=0,
        )

    @pl.when(ph == 1)
    def _():
        inv_n = 1.0 / float(n_nodes)
        mu = stat_ref[0:1, :] * inv_n
        var = stat_ref[1:2, :] * inv_n - mu * mu
        xb = (o2 - mu) / jnp.sqrt(var + 1e-5) * gamma[...] + beta[...]
        y = _softplus(
            jnp.dot(_softplus(xb), wnl[...], preferred_element_type=jnp.float32)
            + bnl[...]
        )
        out_ref[...] = y + skip[...]


GRP = 4


def _sc_gather(table, idxp, R, width, grp=GRP):
    rows_per = idxp.shape[0] // NW
    npairs = rows_per // (2 * grp)
    mesh = plsc.VectorSubcoreMesh(core_axis_name="c", subcore_axis_name="s")

    @functools.partial(
        pl.kernel,
        out_type=jax.ShapeDtypeStruct((R * CHUNK, width), jnp.float32),
        mesh=mesh,
        scratch_types=[
            pltpu.VMEM((rows_per, CHUNK), jnp.int32),
            pltpu.VMEM((2 * grp, CHUNK, width), jnp.float32),
            pltpu.SemaphoreType.DMA,
            pltpu.SemaphoreType.DMA,
            pltpu.SemaphoreType.DMA,
            pltpu.SemaphoreType.DMA,
        ],
        compiler_params=pltpu.CompilerParams(use_tc_tiling_on_sc=False),
    )
    def gk(table_hbm, idx_hbm, out_hbm, idx_all, bufs, sg0, sg1, ss0, ss1):
        cid = lax.axis_index("c")
        sid = lax.axis_index("s")
        wid = sid * 2 + cid
        base = wid * rows_per
        pltpu.sync_copy(idx_hbm.at[pl.ds(base, rows_per)], idx_all)
        sgs = (sg0, sg1)
        sss = (ss0, ss1)

        def gather_desc(g, q, s):
            j = g * grp + q
            return pltpu.make_async_copy(
                table_hbm.at[idx_all.at[j]], bufs.at[s * grp + q], sgs[s]
            )

        def store_desc(g, q, s):
            j = g * grp + q
            return pltpu.make_async_copy(
                bufs.at[s * grp + q],
                out_hbm.at[pl.ds((base + j) * CHUNK, CHUNK)],
                sss[s],
            )

        def live(g, q):
            return base + g * grp + q < R

        def body(jp, carry):
            g0 = jp * 2
            g1 = g0 + 1
            for q in range(grp):
                @pl.when(jnp.logical_and(g0 >= 2, live(g0 - 2, q)))
                def _(q=q):
                    store_desc(g0 - 2, q, 0).wait()
            for q in range(grp):
                @pl.when(live(g0, q))
                def _(q=q):
                    gather_desc(g0, q, 0).start()
            for q in range(grp):
                @pl.when(jnp.logical_and(g0 >= 2, live(g0 - 1, q)))
                def _(q=q):
                    store_desc(g0 - 1, q, 1).wait()
            for q in range(grp):
                @pl.when(live(g1, q))
                def _(q=q):
                    gather_desc(g1, q, 1).start()
            for q in range(grp):
                @pl.when(live(g0, q))
                def _(q=q):
                    gather_desc(g0, q, 0).wait()
                    store_desc(g0, q, 0).start()
            for q in range(grp):
                @pl.when(live(g1, q))
                def _(q=q):
                    gather_desc(g1, q, 1).wait()
                    store_desc(g1, q, 1).start()
            return carry

        lax.fori_loop(0, npairs, body, 0)
        gl0 = (npairs - 1) * 2
        for q in range(grp):
            @pl.when(live(gl0, q))
            def _(q=q):
                store_desc(gl0, q, 0).wait()
        for q in range(grp):
            @pl.when(live(gl0 + 1, q))
            def _(q=q):
                store_desc(gl0 + 1, q, 1).wait()

    return gk(table, idxp)


def _sc_scatter(msg, idxp, R, n_nodes, width, with_count, grp=GRP):
    rows_per = idxp.shape[0] // NW
    npairs = rows_per // (2 * grp)
    npsub = n_nodes // 16
    mesh = plsc.VectorSubcoreMesh(core_axis_name="c", subcore_axis_name="s")

    zacc = jnp.zeros((n_nodes, width), jnp.float32)
    out_type = [jax.ShapeDtypeStruct((2, n_nodes, width), jnp.float32)]
    scratch = [
        pltpu.VMEM((rows_per, CHUNK), jnp.int32),
        pltpu.VMEM((2 * grp, CHUNK, width), jnp.float32),
        pltpu.VMEM_SHARED((n_nodes, width), jnp.float32),
    ]
    if with_count:
        zcnt = jnp.zeros((n_nodes, 8), jnp.float32)
        ones = jnp.ones((CHUNK, 8), jnp.float32)
        out_type.append(jax.ShapeDtypeStruct((2, n_nodes, 8), jnp.float32))
        scratch += [
            pltpu.VMEM((CHUNK, 8), jnp.float32),
            pltpu.VMEM_SHARED((n_nodes, 8), jnp.float32),
        ]
    scratch += [pltpu.SemaphoreType.DMA] * 6

    @functools.partial(
        pl.kernel, out_type=tuple(out_type), mesh=mesh, scratch_types=scratch,
        compiler_params=pltpu.CompilerParams(use_tc_tiling_on_sc=False),
    )
    def sk(*refs):
        if with_count:
            (m_hbm, idx_hbm, za_hbm, zc_hbm, ones_hbm, accp_hbm, cntp_hbm,
             idx_all, bufs, acc_sh, ones_v, cnt_sh,
             sl0, sl1, sa0, sa1, sc0, sc1) = refs
        else:
            (m_hbm, idx_hbm, za_hbm, accp_hbm,
             idx_all, bufs, acc_sh, sl0, sl1, sa0, sa1, sc0, sc1) = refs
        cid = lax.axis_index("c")
        sid = lax.axis_index("s")
        wid = sid * 2 + cid
        base = wid * rows_per

        pltpu.sync_copy(
            za_hbm.at[pl.ds(sid * npsub, npsub)],
            acc_sh.at[pl.ds(sid * npsub, npsub)],
        )
        if with_count:
            pltpu.sync_copy(
                zc_hbm.at[pl.ds(sid * npsub, npsub)],
                cnt_sh.at[pl.ds(sid * npsub, npsub)],
            )
            pltpu.sync_copy(ones_hbm, ones_v)
        pltpu.sync_copy(idx_hbm.at[pl.ds(base, rows_per)], idx_all)
        plsc.subcore_barrier()

        sls = (sl0, sl1)
        sas = (sa0, sa1)
        scs = (sc0, sc1)

        def load_desc(g, q, s):
            j = g * grp + q
            return pltpu.make_async_copy(
                m_hbm.at[pl.ds((base + j) * CHUNK, CHUNK)],
                bufs.at[s * grp + q],
                sls[s],
            )

        def add_desc(g, q, s):
            j = g * grp + q
            return pltpu.make_async_copy(
                bufs.at[s * grp + q], acc_sh.at[idx_all.at[j]], sas[s]
            )

        def cnt_desc(g, q, s):
            j = g * grp + q
            return pltpu.make_async_copy(
                ones_v, cnt_sh.at[idx_all.at[j]], scs[s]
            )

        def live(g, q):
            return base + g * grp + q < R

        def body(jp, carry):
            g0 = jp * 2
            g1 = g0 + 1
            for q in range(grp):
                @pl.when(jnp.logical_and(g0 >= 2, live(g0 - 2, q)))
                def _(q=q):
                    add_desc(g0 - 2, q, 0).wait()
                    if with_count:
                        cnt_desc(g0 - 2, q, 0).wait()
            for q in range(grp):
                @pl.when(live(g0, q))
                def _(q=q):
                    load_desc(g0, q, 0).start()
            for q in range(grp):
                @pl.when(jnp.logical_and(g0 >= 2, live(g0 - 1, q)))
                def _(q=q):
                    add_desc(g0 - 1, q, 1).wait()
                    if with_count:
                        cnt_desc(g0 - 1, q, 1).wait()
            for q in range(grp):
                @pl.when(live(g1, q))
                def _(q=q):
                    load_desc(g1, q, 1).start()
            for q in range(grp):
                @pl.when(live(g0, q))
                def _(q=q):
                    load_desc(g0, q, 0).wait()
                    add_desc(g0, q, 0).start(add=True)
                    if with_count:
                        cnt_desc(g0, q, 0).start(add=True)
            for q in range(grp):
                @pl.when(live(g1, q))
                def _(q=q):
                    load_desc(g1, q, 1).wait()
                    add_desc(g1, q, 1).start(add=True)
                    if with_count:
                        cnt_desc(g1, q, 1).start(add=True)
            return carry

        lax.fori_loop(0, npairs, body, 0)
        gl0 = (npairs - 1) * 2
        for q in range(grp):
            @pl.when(live(gl0, q))
            def _(q=q):
                add_desc(gl0, q, 0).wait()
                if with_count:
                    cnt_desc(gl0, q, 0).wait()
        for q in range(grp):
            @pl.when(live(gl0 + 1, q))
            def _(q=q):
                add_desc(gl0 + 1, q, 1).wait()
                if with_count:
                    cnt_desc(gl0 + 1, q, 1).wait()
        plsc.subcore_barrier()

        pltpu.sync_copy(
            acc_sh.at[pl.ds(sid * npsub, npsub)],
            accp_hbm.at[cid, pl.ds(sid * npsub, npsub)],
        )
        if with_count:
            pltpu.sync_copy(
                cnt_sh.at[pl.ds(sid * npsub, npsub)],
                cntp_hbm.at[cid, pl.ds(sid * npsub, npsub)],
            )

    if with_count:
        return sk(msg, idxp, zacc, zcnt, ones)
    return sk(msg, idxp, zacc)[0]


def _build_p1(fc1_W2, fc1_b2):
    w0 = fc1_W2[:, :256].reshape(16, 16, 16)
    w1 = fc1_W2[:, 256:384].reshape(16, 16, 8)
    w2 = fc1_W2[:, 384:512].reshape(16, 16, 8)
    p = jnp.concatenate([w0, w1, w2], axis=2)
    p = p.transpose(1, 0, 2).reshape(16, 512)
    b = jnp.concatenate(
        [
            fc1_b2[:256].reshape(16, 16),
            fc1_b2[256:384].reshape(16, 8),
            fc1_b2[384:512].reshape(16, 8),
        ],
        axis=1,
    )
    return jnp.concatenate([p, b], axis=1)


def _build_p2(fc2_W2, fc2_b2):
    w00 = fc2_W2[:, :256].reshape(16, 16, 16)
    w11 = fc2_W2[:, 256:384].reshape(16, 8, 16)
    w22 = fc2_W2[:, 384:512].reshape(16, 8, 16)
    p = jnp.concatenate([w00, w11, w22], axis=1)
    p = p.transpose(1, 0, 2).reshape(32, 256)
    b = jnp.concatenate(
        [
            fc2_b2[:256].reshape(16, 16),
            fc2_b2[256:384].reshape(8, 16),
            fc2_b2[384:512].reshape(8, 16),
        ],
        axis=0,
    )
    return jnp.concatenate([p, b], axis=1)


def kernel(node_feature, edge_index, edge_feature, edge_vec, W_node, b_node,
           W_skip, b_skip, fc1_W1, fc1_b1, fc1_W2, fc1_b2, fc2_W1, fc2_b1,
           fc2_W2, fc2_b2, bn_gamma, bn_beta, W_nl2, b_nl2):
    N, D = node_feature.shape
    E = edge_index.shape[1]
    R = E // CHUNK

    rows_per = -(-R // NW)
    rows_per += (-rows_per) % (2 * GRP)
    pad = NW * rows_per - R
    src2d = jnp.pad(edge_index[0].reshape(R, CHUNK).astype(jnp.int32),
                    ((0, pad), (0, 0)))
    dst2d = jnp.pad(edge_index[1].reshape(R, CHUNK).astype(jnp.int32),
                    ((0, pad), (0, 0)))

    p1t = _build_p1(fc1_W2, fc1_b2).T
    p2t = _build_p2(fc2_W2, fc2_b2).T
    w1at = fc1_W1.T
    w2at = fc2_W1.T
    bn2 = b_node.reshape(1, NSC)
    bs2 = b_skip.reshape(1, D)
    b1c = fc1_b1.reshape(16, 1)
    b2c = fc2_b1.reshape(16, 1)
    gamma = bn_gamma.reshape(1, NSC)
    beta = bn_beta.reshape(1, NSC)
    bnl = b_nl2.reshape(1, D)

    BN_ = 2000
    h, skip = pl.pallas_call(
        _node_body,
        grid=(N // BN_,),
        in_specs=[
            pl.BlockSpec((BN_, D), lambda i: (i, 0)),
            pl.BlockSpec((D, NSC), lambda i: (0, 0)),
            pl.BlockSpec((1, NSC), lambda i: (0, 0)),
            pl.BlockSpec((D, D), lambda i: (0, 0)),
            pl.BlockSpec((1, D), lambda i: (0, 0)),
        ],
        out_specs=[
            pl.BlockSpec((BN_, NSC), lambda i: (i, 0)),
            pl.BlockSpec((BN_, D), lambda i: (i, 0)),
        ],
        out_shape=[
            jax.ShapeDtypeStruct((N, NSC), jnp.float32),
            jax.ShapeDtypeStruct((N, D), jnp.float32),
        ],
    )(node_feature, W_node, bn2, W_skip, bs2)

    xh = _sc_gather(h, dst2d, R, NSC, grp=5)

    BE = 8000
    m1 = pl.pallas_call(
        _edge1_body,
        grid=(E // BE,),
        in_specs=[
            pl.BlockSpec((BE, 16), lambda i: (i, 0)),
            pl.BlockSpec((BE, 3), lambda i: (i, 0)),
            pl.BlockSpec((BE, NSC), lambda i: (i, 0)),
            pl.BlockSpec((16, 16), lambda i: (0, 0)),
            pl.BlockSpec((16, 1), lambda i: (0, 0)),
            pl.BlockSpec((544, 16), lambda i: (0, 0)),
        ],
        out_specs=pl.BlockSpec((BE, F1), lambda i: (i, 0)),
        out_shape=jax.ShapeDtypeStruct((E, F1), jnp.float32),
    )(edge_feature, edge_vec, xh, w1at, b1c, p1t)

    o1p, cntp = _sc_scatter(m1, src2d, R, N, F1, with_count=True, grp=2)

    BF = 2000
    o1 = pl.pallas_call(
        _fin1_body,
        grid=(N // BF,),
        in_specs=[
            pl.BlockSpec((2, BF, F1), lambda i: (0, i, 0)),
            pl.BlockSpec((2, BF, 8), lambda i: (0, i, 0)),
            pl.BlockSpec((BF, NSC), lambda i: (i, 0)),
        ],
        out_specs=pl.BlockSpec((BF, F1), lambda i: (i, 0)),
        out_shape=jax.ShapeDtypeStruct((N, F1), jnp.float32),
    )(o1p, cntp, h)

    xo = _sc_gather(o1, dst2d, R, F1, grp=5)

    m2 = pl.pallas_call(
        _edge2_body,
        grid=(E // BE,),
        in_specs=[
            pl.BlockSpec((BE, 16), lambda i: (i, 0)),
            pl.BlockSpec((BE, 3), lambda i: (i, 0)),
            pl.BlockSpec((BE, F1), lambda i: (i, 0)),
            pl.BlockSpec((16, 16), lambda i: (0, 0)),
            pl.BlockSpec((16, 1), lambda i: (0, 0)),
            pl.BlockSpec((272, 32), lambda i: (0, 0)),
        ],
        out_specs=pl.BlockSpec((BE, NSC), lambda i: (i, 0)),
        out_shape=jax.ShapeDtypeStruct((E, NSC), jnp.float32),
    )(edge_feature, edge_vec, xo, w2at, b2c, p2t)

    o2p = _sc_scatter(m2, src2d, R, N, NSC, with_count=False, grp=5)

    BP = 2000
    out = pl.pallas_call(
        functools.partial(_epi_body, N),
        grid=(2, N // BP),
        in_specs=[
            pl.BlockSpec((2, BP, NSC), lambda p, i: (0, i, 0)),
            pl.BlockSpec((2, BP, 8), lambda p, i: (0, i, 0)),
            pl.BlockSpec((BP, D), lambda p, i: (i, 0)),
            pl.BlockSpec((1, NSC), lambda p, i: (0, 0)),
            pl.BlockSpec((1, NSC), lambda p, i: (0, 0)),
            pl.BlockSpec((NSC, D), lambda p, i: (0, 0)),
            pl.BlockSpec((1, D), lambda p, i: (0, 0)),
        ],
        out_specs=pl.BlockSpec((BP, D), lambda p, i: (i, 0)),
        out_shape=jax.ShapeDtypeStruct((N, D), jnp.float32),
        scratch_shapes=[pltpu.VMEM((2, NSC), jnp.float32)],
    )(o2p, cntp, skip, gamma, beta, W_nl2, bnl)

    return out
